# Initial kernel scaffold; baseline (speedup 1.0000x reference)
import jax
import jax.numpy as jnp
from jax.experimental import pallas as pl


def kernel(x, hyperedge_index, r, W1, b1, W2, b2):
    raise NotImplementedError("write your pallas kernel here")


# speedup vs baseline: 1.0990x; 1.0990x over previous
import jax
import jax.numpy as jnp
from jax.experimental import pallas as pl
from jax.experimental.pallas import tpu as pltpu

N_NODES_C = 10000
N_HE_C = 10000


def _mm_kernel(x_ref, w_ref, b_ref, o_ref):
    o_ref[...] = jnp.dot(x_ref[...], w_ref[...],
                         preferred_element_type=jnp.float32) + b_ref[...]


def _matmul_bias(x, w, b):
    n, k = x.shape
    m = w.shape[1]
    blk = 2000
    grid = (n // blk,)
    return pl.pallas_call(
        _mm_kernel,
        grid=grid,
        in_specs=[
            pl.BlockSpec((blk, k), lambda i: (i, 0)),
            pl.BlockSpec((k, m), lambda i: (0, 0)),
            pl.BlockSpec((m,), lambda i: (0,)),
        ],
        out_specs=pl.BlockSpec((blk, m), lambda i: (i, 0)),
        out_shape=jax.ShapeDtypeStruct((n, m), jnp.float32),
    )(x, w, b)


def _build_laplacian(x, hyperedge_index, r):
    node_idx = hyperedge_index[0]
    he_idx = hyperedge_index[1]
    s = x @ r
    sv = s[node_idx]
    seg_max = jax.ops.segment_max(sv, he_idx, num_segments=N_HE_C)
    seg_min = jax.ops.segment_min(sv, he_idx, num_segments=N_HE_C)
    big = jnp.int32(N_NODES_C)
    cand_hi = jnp.where(sv >= seg_max[he_idx], node_idx, big)
    u_hi = jax.ops.segment_min(cand_hi, he_idx, num_segments=N_HE_C)
    cand_lo = jnp.where(sv <= seg_min[he_idx], node_idx, big)
    u_lo = jax.ops.segment_min(cand_lo, he_idx, num_segments=N_HE_C)
    valid = (u_hi < N_NODES_C) & (u_lo < N_NODES_C) & (u_hi != u_lo)
    src = jnp.where(valid, u_hi, 0)
    dst = jnp.where(valid, u_lo, 0)
    w = valid.astype(jnp.float32)
    loop = jnp.arange(N_NODES_C, dtype=jnp.int32)
    row = jnp.concatenate([src, dst, loop])
    col = jnp.concatenate([dst, src, loop])
    ww = jnp.concatenate([w, w, jnp.ones((N_NODES_C,), jnp.float32)])
    deg = jnp.zeros((N_NODES_C,), jnp.float32).at[row].add(ww)
    dinv = jnp.where(deg > 0, jax.lax.rsqrt(deg), 0.0)
    wn = dinv[row] * ww * dinv[col]
    return row, col, wn


def _spmm(row, col, wn, X):
    return jnp.zeros((N_NODES_C, X.shape[1]), X.dtype).at[row].add(wn[:, None] * X[col])


def kernel(x, hyperedge_index, r, W1, b1, W2, b2):
    row, col, wn = _build_laplacian(x, hyperedge_index, r)
    h = _matmul_bias(x, W1, b1)
    h = _spmm(row, col, wn, h)
    h = jax.nn.relu(h)
    o = _matmul_bias(h, W2, b2)
    o = _spmm(row, col, wn, o)
    return o


# speedup vs baseline: 15.3773x; 13.9925x over previous
import functools

import jax
import jax.numpy as jnp
from jax import lax
from jax.experimental import pallas as pl
from jax.experimental.pallas import tpu as pltpu
from jax.experimental.pallas import tpu_sc as plsc

N_NODES_C = 10000
N_HE_C = 10000
NNZ_C = 320000
NP = 10240
ST = 640
EPT = NNZ_C // 16
EPH = EPT // 2
UPT = 2 * ST
NUPD = 16 * UPT
CH = 128
NCH = UPT // CH
BIG = N_NODES_C
NEGF = -3.0e38
POSF = 3.0e38

_mesh = plsc.VectorSubcoreMesh(core_axis_name="c", subcore_axis_name="s")


def _mm_kernel(x_ref, w_ref, b_ref, o_ref):
    o_ref[...] = jnp.dot(x_ref[...], w_ref[...],
                         preferred_element_type=jnp.float32) + b_ref[...]


def _matmul_bias(x, w, b):
    n, k = x.shape
    m = w.shape[1]
    blk = 2000
    return pl.pallas_call(
        _mm_kernel,
        grid=(n // blk,),
        in_specs=[
            pl.BlockSpec((blk, k), lambda i: (i, 0)),
            pl.BlockSpec((k, m), lambda i: (0, 0)),
            pl.BlockSpec((m,), lambda i: (0,)),
        ],
        out_specs=pl.BlockSpec((blk, m), lambda i: (i, 0)),
        out_shape=jax.ShapeDtypeStruct((n, m), jnp.float32),
    )(x, w, b)


def _matvec_kernel(x_ref, r_ref, o_ref):
    o_ref[...] = jnp.dot(x_ref[...], r_ref[...],
                         preferred_element_type=jnp.float32)


def _matvec(x, r):
    n, k = x.shape
    blk = 2000
    return pl.pallas_call(
        _matvec_kernel,
        grid=(n // blk,),
        in_specs=[
            pl.BlockSpec((blk, k), lambda i: (i, 0)),
            pl.BlockSpec((k, 128), lambda i: (0, 0)),
        ],
        out_specs=pl.BlockSpec((blk, 128), lambda i: (i, 0)),
        out_shape=jax.ShapeDtypeStruct((n, 128), jnp.float32),
    )(x, jnp.tile(r[:, None], (1, 128)))[:, 0]


def _fill(ref, nwords, val, dtype):
    vec = jnp.full((16,), val, dtype)

    def body(i, _):
        ref[pl.ds(i * 16, 16)] = vec
        return 0

    lax.fori_loop(0, nwords // 16, body, 0)


def _winner_rmw(conflict_ref, idx, mask0, lane, updates):
    def cond(pend):
        return jnp.any(pend)

    def body(pend):
        plsc.store_scatter(conflict_ref, [idx], lane, mask=pend)
        win = plsc.load_gather(conflict_ref, [idx], mask=pend)
        wm = pend & (win == lane)
        for ref, val, comb in updates:
            cur = plsc.load_gather(ref, [idx], mask=wm)
            plsc.store_scatter(ref, [idx], comb(cur, val), mask=wm)
        return pend & jnp.logical_not(wm)

    lax.while_loop(cond, body, mask0)


def _combine_stripe(sh, stf, t, op, init, nrefs=16):
    for k in range(nrefs):
        pltpu.sync_copy(sh.at[k, pl.ds(t * ST, ST)], stf.at[k])

    def make_body(out_ref):
        def body(j, _):
            acc = jnp.full((16,), init)
            for k in range(nrefs):
                acc = op(acc, stf[k, pl.ds(j * 16, 16)])
            out_ref[pl.ds(j * 16, 16)] = acc
            return 0
        return body

    return make_body


def _gb_body(s_hbm, nidx_hbm, hidx_hbm,
             ro_hbm, ri_hbm, w_hbm, dscale_hbm,
             s_tab, idx_n, idx_h, segA, segB, uA, uB, conflict,
             stf, sti, ustrA, ustrB, srcb, dstb, dinvstr, wbuf,
             shF, shI):
    t = lax.axis_index("s")
    lane = lax.iota(jnp.int32, 16)
    full = jnp.full((16,), True)

    pltpu.sync_copy(s_hbm, s_tab)
    _fill(segA, NP, NEGF, jnp.float32)
    _fill(segB, NP, POSF, jnp.float32)

    def phaseB(i, _):
        hv = idx_h[pl.ds(i * 16, 16)]
        nv = idx_n[pl.ds(i * 16, 16)]
        sv = plsc.load_gather(s_tab, [nv])
        _winner_rmw(conflict, hv, full, lane,
                    [(segA, sv, jnp.maximum), (segB, sv, jnp.minimum)])
        return 0

    for half in range(2):
        pltpu.sync_copy(nidx_hbm.at[pl.ds(t * EPT + half * EPH, EPH)], idx_n)
        pltpu.sync_copy(hidx_hbm.at[pl.ds(t * EPT + half * EPH, EPH)], idx_h)
        lax.fori_loop(0, EPH // 16, phaseB, 0)

    pltpu.sync_copy(segA, shF.at[t])
    plsc.subcore_barrier()
    body = _combine_stripe(shF, stf, t, jnp.maximum, NEGF)(dinvstr)
    lax.fori_loop(0, ST // 16, body, 0)
    pltpu.sync_copy(dinvstr, shF.at[0, pl.ds(t * ST, ST)])
    plsc.subcore_barrier()
    pltpu.sync_copy(shF.at[0], segA)
    plsc.subcore_barrier()

    pltpu.sync_copy(segB, shF.at[t])
    plsc.subcore_barrier()
    body = _combine_stripe(shF, stf, t, jnp.minimum, POSF)(dinvstr)
    lax.fori_loop(0, ST // 16, body, 0)
    pltpu.sync_copy(dinvstr, shF.at[0, pl.ds(t * ST, ST)])
    plsc.subcore_barrier()
    pltpu.sync_copy(shF.at[0], segB)
    plsc.subcore_barrier()

    _fill(uA, NP, BIG, jnp.int32)
    _fill(uB, NP, BIG, jnp.int32)

    def phaseC(i, _):
        hv = idx_h[pl.ds(i * 16, 16)]
        nv = idx_n[pl.ds(i * 16, 16)]
        sv = plsc.load_gather(s_tab, [nv])
        mx = plsc.load_gather(segA, [hv])
        mn = plsc.load_gather(segB, [hv])
        cand_hi = jnp.where(sv == mx, nv, BIG)
        cand_lo = jnp.where(sv == mn, nv, BIG)
        _winner_rmw(conflict, hv, full, lane,
                    [(uA, cand_hi, jnp.minimum), (uB, cand_lo, jnp.minimum)])
        return 0

    for half in range(2):
        pltpu.sync_copy(nidx_hbm.at[pl.ds(t * EPT + half * EPH, EPH)], idx_n)
        pltpu.sync_copy(hidx_hbm.at[pl.ds(t * EPT + half * EPH, EPH)], idx_h)
        lax.fori_loop(0, EPH // 16, phaseC, 0)

    pltpu.sync_copy(uA, shI.at[t])
    plsc.subcore_barrier()
    body = _combine_stripe(shI, sti, t, jnp.minimum, BIG)(ustrA)
    lax.fori_loop(0, ST // 16, body, 0)
    plsc.subcore_barrier()
    pltpu.sync_copy(uB, shI.at[t])
    plsc.subcore_barrier()
    body = _combine_stripe(shI, sti, t, jnp.minimum, BIG)(ustrB)
    lax.fori_loop(0, ST // 16, body, 0)
    plsc.subcore_barrier()

    def phaseD1(j, _):
        ua = ustrA[pl.ds(j * 16, 16)]
        ub = ustrB[pl.ds(j * 16, 16)]
        valid = (ua < BIG) & (ub < BIG) & (ua != ub)
        srcb[pl.ds(j * 16, 16)] = jnp.where(valid, ua, 0)
        dstb[pl.ds(j * 16, 16)] = jnp.where(valid, ub, 0)
        return 0

    lax.fori_loop(0, ST // 16, phaseD1, 0)

    _fill(segA, NP, jnp.float32(0.0), jnp.float32)
    onef = jnp.full((16,), 1.0, jnp.float32)

    def phaseD2(j, _):
        ua = ustrA[pl.ds(j * 16, 16)]
        ub = ustrB[pl.ds(j * 16, 16)]
        sv16 = srcb[pl.ds(j * 16, 16)]
        dv16 = dstb[pl.ds(j * 16, 16)]
        valid = (ua < BIG) & (ub < BIG) & (ua != ub)
        _winner_rmw(conflict, sv16, valid, lane,
                    [(segA, onef, lambda c, v: c + v)])
        _winner_rmw(conflict, dv16, valid, lane,
                    [(segA, onef, lambda c, v: c + v)])
        return 0

    lax.fori_loop(0, ST // 16, phaseD2, 0)

    pltpu.sync_copy(segA, shF.at[t])
    plsc.subcore_barrier()

    def degbody(j, _):
        acc = jnp.full((16,), 0.0, jnp.float32)
        for k in range(16):
            acc = acc + stf[k, pl.ds(j * 16, 16)]
        deg = acc + 1.0
        bits = plsc.bitcast(deg, jnp.int32)
        y = plsc.bitcast(jnp.int32(0x5F3759DF) - (bits >> 1), jnp.float32)
        for _i in range(3):
            y = y * (1.5 - 0.5 * deg * y * y)
        dinvstr[pl.ds(j * 16, 16)] = y
        return 0

    lax.fori_loop(0, ST // 16, degbody, 0)
    pltpu.sync_copy(dinvstr, shF.at[0, pl.ds(t * ST, ST)])
    plsc.subcore_barrier()
    pltpu.sync_copy(shF.at[0], s_tab)
    plsc.subcore_barrier()

    def dsbody(j, _):
        y = dinvstr[pl.ds(j * 16, 16)]
        wbuf[pl.ds(j * 16, 16)] = y * y
        return 0

    lax.fori_loop(0, ST // 16, dsbody, 0)
    pltpu.sync_copy(wbuf.at[pl.ds(0, ST)], dscale_hbm.at[pl.ds(t * ST, ST)])

    def wbody(j, _):
        ua = ustrA[pl.ds(j * 16, 16)]
        ub = ustrB[pl.ds(j * 16, 16)]
        sv16 = srcb[pl.ds(j * 16, 16)]
        dv16 = dstb[pl.ds(j * 16, 16)]
        valid = (ua < BIG) & (ub < BIG) & (ua != ub)
        ds_ = plsc.load_gather(s_tab, [sv16])
        dd_ = plsc.load_gather(s_tab, [dv16])
        wv = jnp.where(valid, ds_ * dd_, 0.0)
        wbuf[pl.ds(j * 16, 16)] = wv
        wbuf[pl.ds(ST + j * 16, 16)] = wv
        return 0

    lax.fori_loop(0, ST // 16, wbody, 0)

    pltpu.sync_copy(srcb, ro_hbm.at[pl.ds(t * UPT, ST)])
    pltpu.sync_copy(dstb, ro_hbm.at[pl.ds(t * UPT + ST, ST)])
    pltpu.sync_copy(dstb, ri_hbm.at[pl.ds(t * UPT, ST)])
    pltpu.sync_copy(srcb, ri_hbm.at[pl.ds(t * UPT + ST, ST)])
    pltpu.sync_copy(wbuf, w_hbm.at[pl.ds(t * UPT, UPT)])


def _gb_kernel_body(s_hbm, nidx_hbm, hidx_hbm,
                    ro_hbm, ri_hbm, w_hbm, dscale_hbm, *scratch):
    c = lax.axis_index("c")
    pl.when(c == 0)(lambda: _gb_body(
        s_hbm, nidx_hbm, hidx_hbm, ro_hbm, ri_hbm, w_hbm, dscale_hbm,
        *scratch))


_graph_build = functools.partial(
    pl.kernel,
    out_type=[
        jax.ShapeDtypeStruct((NUPD,), jnp.int32),
        jax.ShapeDtypeStruct((NUPD,), jnp.int32),
        jax.ShapeDtypeStruct((NUPD,), jnp.float32),
        jax.ShapeDtypeStruct((NP,), jnp.float32),
    ],
    mesh=_mesh,
    compiler_params=pltpu.CompilerParams(needs_layout_passes=False),
    scratch_types=[
        pltpu.VMEM((NP,), jnp.float32),
        pltpu.VMEM((EPH,), jnp.int32),
        pltpu.VMEM((EPH,), jnp.int32),
        pltpu.VMEM((NP,), jnp.float32),
        pltpu.VMEM((NP,), jnp.float32),
        pltpu.VMEM((NP,), jnp.int32),
        pltpu.VMEM((NP,), jnp.int32),
        pltpu.VMEM((NP,), jnp.int32),
        pltpu.VMEM((16, ST), jnp.float32),
        pltpu.VMEM((16, ST), jnp.int32),
        pltpu.VMEM((ST,), jnp.int32),
        pltpu.VMEM((ST,), jnp.int32),
        pltpu.VMEM((ST,), jnp.int32),
        pltpu.VMEM((ST,), jnp.int32),
        pltpu.VMEM((ST,), jnp.float32),
        pltpu.VMEM((UPT,), jnp.float32),
        pltpu.VMEM_SHARED((16, NP), jnp.float32),
        pltpu.VMEM_SHARED((16, NP), jnp.int32),
    ],
)(_gb_kernel_body)


def kernel(x, hyperedge_index, r, W1, b1, W2, b2):
    node_idx = hyperedge_index[0]
    he_idx = hyperedge_index[1]
    s = _matvec(x, r)
    s_pad = jnp.pad(s, (0, NP - N_NODES_C))
    ro, ri, w, dsc = _graph_build(s_pad, node_idx, he_idx)

    h = _matmul_bias(x, W1, b1)
    hp = jnp.pad(h, ((0, NP - N_NODES_C), (0, 0)))
    h1 = (jnp.zeros((NP, h.shape[1]), jnp.float32).at[ro].add(w[:, None] * hp[ri])
          + dsc[:, None] * hp)
    h1 = jax.nn.relu(h1[:N_NODES_C])

    o = _matmul_bias(h1, W2, b2)
    op_ = jnp.pad(o, ((0, NP - N_NODES_C), (0, 0)))
    o2 = (jnp.zeros((NP, o.shape[1]), jnp.float32).at[ro].add(w[:, None] * op_[ri])
          + dsc[:, None] * op_)
    return o2[:N_NODES_C]


# speedup vs baseline: 22.7159x; 1.4772x over previous
import functools

import jax
import jax.numpy as jnp
from jax import lax
from jax.experimental import pallas as pl
from jax.experimental.pallas import tpu as pltpu
from jax.experimental.pallas import tpu_sc as plsc

N_NODES_C = 10000
N_HE_C = 10000
NNZ_C = 320000
NP = 10240
ST = 640
EPT = NNZ_C // 16
EPH = EPT // 2
UPT = 2 * ST
NUPD = 16 * UPT
CH = 128
NCH = UPT // CH
BIG = N_NODES_C
NEGF = -3.0e38
POSF = 3.0e38

_mesh = plsc.VectorSubcoreMesh(core_axis_name="c", subcore_axis_name="s")


def _mm_kernel(x_ref, w_ref, b_ref, o_ref):
    o_ref[...] = jnp.dot(x_ref[...], w_ref[...],
                         preferred_element_type=jnp.float32) + b_ref[...]


def _matmul_bias(x, w, b):
    n, k = x.shape
    m = w.shape[1]
    blk = 2000
    return pl.pallas_call(
        _mm_kernel,
        grid=(n // blk,),
        in_specs=[
            pl.BlockSpec((blk, k), lambda i: (i, 0)),
            pl.BlockSpec((k, m), lambda i: (0, 0)),
            pl.BlockSpec((m,), lambda i: (0,)),
        ],
        out_specs=pl.BlockSpec((blk, m), lambda i: (i, 0)),
        out_shape=jax.ShapeDtypeStruct((n, m), jnp.float32),
    )(x, w, b)


def _matvec_kernel(x_ref, r_ref, o_ref):
    o_ref[...] = jnp.dot(x_ref[...], r_ref[...],
                         preferred_element_type=jnp.float32)


def _matvec(x, r):
    n, k = x.shape
    blk = 2000
    return pl.pallas_call(
        _matvec_kernel,
        grid=(n // blk,),
        in_specs=[
            pl.BlockSpec((blk, k), lambda i: (i, 0)),
            pl.BlockSpec((k, 128), lambda i: (0, 0)),
        ],
        out_specs=pl.BlockSpec((blk, 128), lambda i: (i, 0)),
        out_shape=jax.ShapeDtypeStruct((n, 128), jnp.float32),
    )(x, jnp.tile(r[:, None], (1, 128)))[:, 0]


def _fill(ref, nwords, val, dtype):
    vec = jnp.full((16,), val, dtype)

    def body(i, _):
        ref[pl.ds(i * 16, 16)] = vec
        return 0

    lax.fori_loop(0, nwords // 16, body, 0)


def _winner_rmw(conflict_ref, idx, mask0, lane, updates):
    def cond(pend):
        return jnp.any(pend)

    def body(pend):
        plsc.store_scatter(conflict_ref, [idx], lane, mask=pend)
        win = plsc.load_gather(conflict_ref, [idx], mask=pend)
        wm = pend & (win == lane)
        for ref, val, comb in updates:
            cur = plsc.load_gather(ref, [idx], mask=wm)
            plsc.store_scatter(ref, [idx], comb(cur, val), mask=wm)
        return pend & jnp.logical_not(wm)

    lax.while_loop(cond, body, mask0)


def _combine_stripe(sh, stf, t, op, init, nrefs=16):
    for k in range(nrefs):
        pltpu.sync_copy(sh.at[k, pl.ds(t * ST, ST)], stf.at[k])

    def make_body(out_ref):
        def body(j, _):
            acc = jnp.full((16,), init)
            for k in range(nrefs):
                acc = op(acc, stf[k, pl.ds(j * 16, 16)])
            out_ref[pl.ds(j * 16, 16)] = acc
            return 0
        return body

    return make_body


def _gb_body(s_hbm, nidx_hbm, hidx_hbm,
             ro_hbm, ri_hbm, w_hbm, dscale_hbm,
             s_tab, idx_n, idx_h, segA, segB, uA, uB, conflict,
             stf, sti, ustrA, ustrB, srcb, dstb, dinvstr, wbuf,
             shF, shI):
    t = lax.axis_index("s")
    lane = lax.iota(jnp.int32, 16)
    full = jnp.full((16,), True)

    pltpu.sync_copy(s_hbm, s_tab)
    _fill(segA, NP, NEGF, jnp.float32)
    _fill(segB, NP, POSF, jnp.float32)

    def phaseB(i, _):
        hv = idx_h[pl.ds(i * 16, 16)]
        nv = idx_n[pl.ds(i * 16, 16)]
        sv = plsc.load_gather(s_tab, [nv])
        _winner_rmw(conflict, hv, full, lane,
                    [(segA, sv, jnp.maximum), (segB, sv, jnp.minimum)])
        return 0

    for half in range(2):
        pltpu.sync_copy(nidx_hbm.at[pl.ds(t * EPT + half * EPH, EPH)], idx_n)
        pltpu.sync_copy(hidx_hbm.at[pl.ds(t * EPT + half * EPH, EPH)], idx_h)
        lax.fori_loop(0, EPH // 16, phaseB, 0)

    pltpu.sync_copy(segA, shF.at[t])
    plsc.subcore_barrier()
    body = _combine_stripe(shF, stf, t, jnp.maximum, NEGF)(dinvstr)
    lax.fori_loop(0, ST // 16, body, 0)
    pltpu.sync_copy(dinvstr, shF.at[0, pl.ds(t * ST, ST)])
    plsc.subcore_barrier()
    pltpu.sync_copy(shF.at[0], segA)
    plsc.subcore_barrier()

    pltpu.sync_copy(segB, shF.at[t])
    plsc.subcore_barrier()
    body = _combine_stripe(shF, stf, t, jnp.minimum, POSF)(dinvstr)
    lax.fori_loop(0, ST // 16, body, 0)
    pltpu.sync_copy(dinvstr, shF.at[0, pl.ds(t * ST, ST)])
    plsc.subcore_barrier()
    pltpu.sync_copy(shF.at[0], segB)
    plsc.subcore_barrier()

    _fill(uA, NP, BIG, jnp.int32)
    _fill(uB, NP, BIG, jnp.int32)

    def phaseC(i, _):
        hv = idx_h[pl.ds(i * 16, 16)]
        nv = idx_n[pl.ds(i * 16, 16)]
        sv = plsc.load_gather(s_tab, [nv])
        mx = plsc.load_gather(segA, [hv])
        mn = plsc.load_gather(segB, [hv])
        cand_hi = jnp.where(sv == mx, nv, BIG)
        cand_lo = jnp.where(sv == mn, nv, BIG)
        _winner_rmw(conflict, hv, full, lane,
                    [(uA, cand_hi, jnp.minimum), (uB, cand_lo, jnp.minimum)])
        return 0

    for half in range(2):
        pltpu.sync_copy(nidx_hbm.at[pl.ds(t * EPT + half * EPH, EPH)], idx_n)
        pltpu.sync_copy(hidx_hbm.at[pl.ds(t * EPT + half * EPH, EPH)], idx_h)
        lax.fori_loop(0, EPH // 16, phaseC, 0)

    pltpu.sync_copy(uA, shI.at[t])
    plsc.subcore_barrier()
    body = _combine_stripe(shI, sti, t, jnp.minimum, BIG)(ustrA)
    lax.fori_loop(0, ST // 16, body, 0)
    plsc.subcore_barrier()
    pltpu.sync_copy(uB, shI.at[t])
    plsc.subcore_barrier()
    body = _combine_stripe(shI, sti, t, jnp.minimum, BIG)(ustrB)
    lax.fori_loop(0, ST // 16, body, 0)
    plsc.subcore_barrier()

    def phaseD1(j, _):
        ua = ustrA[pl.ds(j * 16, 16)]
        ub = ustrB[pl.ds(j * 16, 16)]
        valid = (ua < BIG) & (ub < BIG) & (ua != ub)
        srcb[pl.ds(j * 16, 16)] = jnp.where(valid, ua, 0)
        dstb[pl.ds(j * 16, 16)] = jnp.where(valid, ub, 0)
        return 0

    lax.fori_loop(0, ST // 16, phaseD1, 0)

    _fill(segA, NP, jnp.float32(0.0), jnp.float32)
    onef = jnp.full((16,), 1.0, jnp.float32)

    def phaseD2(j, _):
        ua = ustrA[pl.ds(j * 16, 16)]
        ub = ustrB[pl.ds(j * 16, 16)]
        sv16 = srcb[pl.ds(j * 16, 16)]
        dv16 = dstb[pl.ds(j * 16, 16)]
        valid = (ua < BIG) & (ub < BIG) & (ua != ub)
        _winner_rmw(conflict, sv16, valid, lane,
                    [(segA, onef, lambda c, v: c + v)])
        _winner_rmw(conflict, dv16, valid, lane,
                    [(segA, onef, lambda c, v: c + v)])
        return 0

    lax.fori_loop(0, ST // 16, phaseD2, 0)

    pltpu.sync_copy(segA, shF.at[t])
    plsc.subcore_barrier()

    def degbody(j, _):
        acc = jnp.full((16,), 0.0, jnp.float32)
        for k in range(16):
            acc = acc + stf[k, pl.ds(j * 16, 16)]
        deg = acc + 1.0
        bits = plsc.bitcast(deg, jnp.int32)
        y = plsc.bitcast(jnp.int32(0x5F3759DF) - (bits >> 1), jnp.float32)
        for _i in range(3):
            y = y * (1.5 - 0.5 * deg * y * y)
        dinvstr[pl.ds(j * 16, 16)] = y
        return 0

    lax.fori_loop(0, ST // 16, degbody, 0)
    pltpu.sync_copy(dinvstr, shF.at[0, pl.ds(t * ST, ST)])
    plsc.subcore_barrier()
    pltpu.sync_copy(shF.at[0], s_tab)
    plsc.subcore_barrier()

    def dsbody(j, _):
        y = dinvstr[pl.ds(j * 16, 16)]
        wbuf[pl.ds(j * 16, 16)] = y * y
        return 0

    lax.fori_loop(0, ST // 16, dsbody, 0)
    pltpu.sync_copy(wbuf.at[pl.ds(0, ST)], dscale_hbm.at[pl.ds(t * ST, ST)])

    def wbody(j, _):
        ua = ustrA[pl.ds(j * 16, 16)]
        ub = ustrB[pl.ds(j * 16, 16)]
        sv16 = srcb[pl.ds(j * 16, 16)]
        dv16 = dstb[pl.ds(j * 16, 16)]
        valid = (ua < BIG) & (ub < BIG) & (ua != ub)
        ds_ = plsc.load_gather(s_tab, [sv16])
        dd_ = plsc.load_gather(s_tab, [dv16])
        wv = jnp.where(valid, ds_ * dd_, 0.0)
        wbuf[pl.ds(j * 16, 16)] = wv
        wbuf[pl.ds(ST + j * 16, 16)] = wv
        return 0

    lax.fori_loop(0, ST // 16, wbody, 0)

    pltpu.sync_copy(srcb, ro_hbm.at[pl.ds(t * UPT, ST)])
    pltpu.sync_copy(dstb, ro_hbm.at[pl.ds(t * UPT + ST, ST)])
    pltpu.sync_copy(dstb, ri_hbm.at[pl.ds(t * UPT, ST)])
    pltpu.sync_copy(srcb, ri_hbm.at[pl.ds(t * UPT + ST, ST)])
    pltpu.sync_copy(wbuf, w_hbm.at[pl.ds(t * UPT, UPT)])


def _gb_kernel_body(s_hbm, nidx_hbm, hidx_hbm,
                    ro_hbm, ri_hbm, w_hbm, dscale_hbm, *scratch):
    c = lax.axis_index("c")
    pl.when(c == 0)(lambda: _gb_body(
        s_hbm, nidx_hbm, hidx_hbm, ro_hbm, ri_hbm, w_hbm, dscale_hbm,
        *scratch))


_graph_build = functools.partial(
    pl.kernel,
    out_type=[
        jax.ShapeDtypeStruct((NUPD,), jnp.int32),
        jax.ShapeDtypeStruct((NUPD,), jnp.int32),
        jax.ShapeDtypeStruct((NUPD,), jnp.float32),
        jax.ShapeDtypeStruct((NP,), jnp.float32),
    ],
    mesh=_mesh,
    compiler_params=pltpu.CompilerParams(needs_layout_passes=False),
    scratch_types=[
        pltpu.VMEM((NP,), jnp.float32),
        pltpu.VMEM((EPH,), jnp.int32),
        pltpu.VMEM((EPH,), jnp.int32),
        pltpu.VMEM((NP,), jnp.float32),
        pltpu.VMEM((NP,), jnp.float32),
        pltpu.VMEM((NP,), jnp.int32),
        pltpu.VMEM((NP,), jnp.int32),
        pltpu.VMEM((NP,), jnp.int32),
        pltpu.VMEM((16, ST), jnp.float32),
        pltpu.VMEM((16, ST), jnp.int32),
        pltpu.VMEM((ST,), jnp.int32),
        pltpu.VMEM((ST,), jnp.int32),
        pltpu.VMEM((ST,), jnp.int32),
        pltpu.VMEM((ST,), jnp.int32),
        pltpu.VMEM((ST,), jnp.float32),
        pltpu.VMEM((UPT,), jnp.float32),
        pltpu.VMEM_SHARED((16, NP), jnp.float32),
        pltpu.VMEM_SHARED((16, NP), jnp.int32),
    ],
)(_gb_kernel_body)


def _spmm_work(h_hbm, hinit_hbm, o_hbm, ro_hbm, ri_hbm, w_hbm,
               gbuf, rov, riv, wv, acc, sem, t, f2):
    stripe = pl.ds(t * ST, ST)
    pltpu.sync_copy(hinit_hbm.at[stripe], acc.at[stripe])
    pltpu.sync_copy(ro_hbm.at[t], rov)
    pltpu.sync_copy(ri_hbm.at[t], riv)
    pltpu.sync_copy(w_hbm.at[pl.ds(t * UPT, UPT)], wv)
    plsc.subcore_barrier()

    for chunk in range(NCH):
        pltpu.async_copy(h_hbm.at[riv.at[chunk]], gbuf, sem).wait()

        def scale16(jj, _):
            w16 = wv[pl.ds(chunk * CH + jj * 16, 16)]
            for k16 in range(16):
                sc = w16[k16]
                for cc in range(f2 // 16):
                    col = pl.ds(cc * 16, 16)
                    row = jj * 16 + k16
                    gbuf[row, col] = gbuf[row, col] * sc
            return 0

        lax.fori_loop(0, CH // 16, scale16, 0)
        pltpu.async_copy(gbuf, acc.at[rov.at[chunk]], sem, add=True).wait()

    plsc.subcore_barrier()
    pltpu.sync_copy(acc.at[stripe], o_hbm.at[stripe])


def _make_spmm(f2):
    def body(ha, hb, hia, hib, ro3, ri3, w, oa, ob,
             gbuf, rov, riv, wv, acc, sem):
        c = lax.axis_index("c")
        t = lax.axis_index("s")
        pl.when(c == 0)(lambda: _spmm_work(
            ha, hia, oa, ro3, ri3, w, gbuf, rov, riv, wv, acc, sem, t, f2))
        pl.when(c == 1)(lambda: _spmm_work(
            hb, hib, ob, ro3, ri3, w, gbuf, rov, riv, wv, acc, sem, t, f2))

    return functools.partial(
        pl.kernel,
        out_type=[
            jax.ShapeDtypeStruct((NP, f2), jnp.float32),
            jax.ShapeDtypeStruct((NP, f2), jnp.float32),
        ],
        mesh=_mesh,
        compiler_params=pltpu.CompilerParams(needs_layout_passes=False,
                                             use_tc_tiling_on_sc=False),
        scratch_types=[
            pltpu.VMEM((CH, f2), jnp.float32),
            pltpu.VMEM((NCH, CH), jnp.int32),
            pltpu.VMEM((NCH, CH), jnp.int32),
            pltpu.VMEM((UPT,), jnp.float32),
            pltpu.VMEM_SHARED((NP, f2), jnp.float32),
            pltpu.SemaphoreType.DMA,
        ],
    )(body)


_spmm64 = _make_spmm(64)
_spmm32 = _make_spmm(32)


def _mm_scale_kernel(x_ref, w_ref, b_ref, d_ref, ha, hb, hia, hib):
    h = jnp.dot(x_ref[...], w_ref[...],
                preferred_element_type=jnp.float32) + b_ref[...]
    hi = h * d_ref[...]
    half = h.shape[1] // 2
    ha[...] = h[:, :half]
    hb[...] = h[:, half:]
    hia[...] = hi[:, :half]
    hib[...] = hi[:, half:]


def _mm1(xp, W1, b1, dsc):
    blk = 2048
    m = W1.shape[1]
    half = m // 2
    sds = jax.ShapeDtypeStruct((NP, half), jnp.float32)
    return pl.pallas_call(
        _mm_scale_kernel,
        grid=(NP // blk,),
        in_specs=[
            pl.BlockSpec((blk, 128), lambda i: (i, 0)),
            pl.BlockSpec((128, m), lambda i: (0, 0)),
            pl.BlockSpec((m,), lambda i: (0,)),
            pl.BlockSpec((blk, 1), lambda i: (i, 0)),
        ],
        out_specs=[pl.BlockSpec((blk, half), lambda i: (i, 0))] * 4,
        out_shape=[sds, sds, sds, sds],
    )(xp, W1, b1, dsc[:, None])


def _mm2_kernel(a_ref, b_ref, w_ref, bias_ref, d_ref, oa, ob, oia, oib):
    h = jnp.concatenate([a_ref[...], b_ref[...]], axis=1)
    h = jax.nn.relu(h)
    o = jnp.dot(h, w_ref[...], preferred_element_type=jnp.float32) + bias_ref[...]
    oi = o * d_ref[...]
    half = o.shape[1] // 2
    oa[...] = o[:, :half]
    ob[...] = o[:, half:]
    oia[...] = oi[:, :half]
    oib[...] = oi[:, half:]


def _mm2(h1a, h1b, W2, b2, dsc):
    blk = 2048
    m = W2.shape[1]
    half = m // 2
    sds = jax.ShapeDtypeStruct((NP, half), jnp.float32)
    return pl.pallas_call(
        _mm2_kernel,
        grid=(NP // blk,),
        in_specs=[
            pl.BlockSpec((blk, 64), lambda i: (i, 0)),
            pl.BlockSpec((blk, 64), lambda i: (i, 0)),
            pl.BlockSpec((128, m), lambda i: (0, 0)),
            pl.BlockSpec((m,), lambda i: (0,)),
            pl.BlockSpec((blk, 1), lambda i: (i, 0)),
        ],
        out_specs=[pl.BlockSpec((blk, half), lambda i: (i, 0))] * 4,
        out_shape=[sds, sds, sds, sds],
    )(h1a, h1b, W2, b2, dsc[:, None])


def kernel(x, hyperedge_index, r, W1, b1, W2, b2):
    node_idx = hyperedge_index[0]
    he_idx = hyperedge_index[1]
    s = _matvec(x, r)
    s_pad = jnp.pad(s, (0, NP - N_NODES_C))
    ro, ri, w, dsc = _graph_build(s_pad, node_idx, he_idx)
    ro3 = ro.reshape(16, NCH, CH)
    ri3 = ri.reshape(16, NCH, CH)

    xp = jnp.pad(x, ((0, NP - N_NODES_C), (0, 0)))
    ha, hb, hia, hib = _mm1(xp, W1, b1, dsc)
    o1a, o1b = _spmm64(ha, hb, hia, hib, ro3, ri3, w)
    oa, ob, oia, oib = _mm2(o1a, o1b, W2, b2, dsc)
    qa, qb = _spmm32(oa, ob, oia, oib, ro3, ri3, w)
    return jnp.concatenate([qa[:N_NODES_C], qb[:N_NODES_C]], axis=1)


# speedup vs baseline: 23.6949x; 1.0431x over previous
import functools

import jax
import jax.numpy as jnp
from jax import lax
from jax.experimental import pallas as pl
from jax.experimental.pallas import tpu as pltpu
from jax.experimental.pallas import tpu_sc as plsc

N_NODES_C = 10000
N_HE_C = 10000
NNZ_C = 320000
NP = 10240
ST = 640
EPT = NNZ_C // 16
EPH = EPT // 2
UPT = 2 * ST
NUPD = 16 * UPT
CH = 128
NCH = UPT // CH
BIG = N_NODES_C
NEGF = -3.0e38
POSF = 3.0e38

_mesh = plsc.VectorSubcoreMesh(core_axis_name="c", subcore_axis_name="s")


def _mm_kernel(x_ref, w_ref, b_ref, o_ref):
    o_ref[...] = jnp.dot(x_ref[...], w_ref[...],
                         preferred_element_type=jnp.float32) + b_ref[...]


def _matmul_bias(x, w, b):
    n, k = x.shape
    m = w.shape[1]
    blk = 2000
    return pl.pallas_call(
        _mm_kernel,
        grid=(n // blk,),
        in_specs=[
            pl.BlockSpec((blk, k), lambda i: (i, 0)),
            pl.BlockSpec((k, m), lambda i: (0, 0)),
            pl.BlockSpec((m,), lambda i: (0,)),
        ],
        out_specs=pl.BlockSpec((blk, m), lambda i: (i, 0)),
        out_shape=jax.ShapeDtypeStruct((n, m), jnp.float32),
    )(x, w, b)


def _matvec_kernel(x_ref, r_ref, o_ref):
    o_ref[...] = jnp.dot(x_ref[...], r_ref[...],
                         preferred_element_type=jnp.float32)


def _matvec(x, r):
    n, k = x.shape
    blk = 2000
    return pl.pallas_call(
        _matvec_kernel,
        grid=(n // blk,),
        in_specs=[
            pl.BlockSpec((blk, k), lambda i: (i, 0)),
            pl.BlockSpec((k, 128), lambda i: (0, 0)),
        ],
        out_specs=pl.BlockSpec((blk, 128), lambda i: (i, 0)),
        out_shape=jax.ShapeDtypeStruct((n, 128), jnp.float32),
    )(x, jnp.tile(r[:, None], (1, 128)))[:, 0]


def _fill(ref, nwords, val, dtype):
    vec = jnp.full((16,), val, dtype)

    def body(i, _):
        ref[pl.ds(i * 16, 16)] = vec
        return 0

    lax.fori_loop(0, nwords // 16, body, 0)


def _winner_rmw(conflict_ref, idx, mask0, lane, updates):
    def cond(pend):
        return jnp.any(pend)

    def body(pend):
        plsc.store_scatter(conflict_ref, [idx], lane, mask=pend)
        win = plsc.load_gather(conflict_ref, [idx], mask=pend)
        wm = pend & (win == lane)
        for ref, val, comb in updates:
            cur = plsc.load_gather(ref, [idx], mask=wm)
            plsc.store_scatter(ref, [idx], comb(cur, val), mask=wm)
        return pend & jnp.logical_not(wm)

    lax.while_loop(cond, body, mask0)


def _combine_stripe(sh, stf, t, op, init, nrefs=16):
    for k in range(nrefs):
        pltpu.sync_copy(sh.at[k, pl.ds(t * ST, ST)], stf.at[k])

    def make_body(out_ref):
        def body(j, _):
            acc = jnp.full((16,), init)
            for k in range(nrefs):
                acc = op(acc, stf[k, pl.ds(j * 16, 16)])
            out_ref[pl.ds(j * 16, 16)] = acc
            return 0
        return body

    return make_body


def _gb_body(s_hbm, nidx_hbm, hidx_hbm,
             ro_hbm, ri_hbm, w_hbm, dscale_hbm,
             s_tab, idx_n, idx_h, segA, segB, uA, uB, conflict,
             stf, sti, ustrA, ustrB, srcb, dstb, dinvstr, wbuf,
             shF, shI):
    t = lax.axis_index("s")
    lane = lax.iota(jnp.int32, 16)
    full = jnp.full((16,), True)

    pltpu.sync_copy(s_hbm, s_tab)
    _fill(segA, NP, NEGF, jnp.float32)
    _fill(segB, NP, POSF, jnp.float32)

    def phaseB(i, _):
        hv = idx_h[pl.ds(i * 16, 16)]
        nv = idx_n[pl.ds(i * 16, 16)]
        sv = plsc.load_gather(s_tab, [nv])
        _winner_rmw(conflict, hv, full, lane,
                    [(segA, sv, jnp.maximum), (segB, sv, jnp.minimum)])
        return 0

    for half in range(2):
        pltpu.sync_copy(nidx_hbm.at[pl.ds(t * EPT + half * EPH, EPH)], idx_n)
        pltpu.sync_copy(hidx_hbm.at[pl.ds(t * EPT + half * EPH, EPH)], idx_h)
        lax.fori_loop(0, EPH // 16, phaseB, 0)

    pltpu.sync_copy(segA, shF.at[t])
    plsc.subcore_barrier()
    body = _combine_stripe(shF, stf, t, jnp.maximum, NEGF)(dinvstr)
    lax.fori_loop(0, ST // 16, body, 0)
    pltpu.sync_copy(dinvstr, shF.at[0, pl.ds(t * ST, ST)])
    plsc.subcore_barrier()
    pltpu.sync_copy(shF.at[0], segA)
    plsc.subcore_barrier()

    pltpu.sync_copy(segB, shF.at[t])
    plsc.subcore_barrier()
    body = _combine_stripe(shF, stf, t, jnp.minimum, POSF)(dinvstr)
    lax.fori_loop(0, ST // 16, body, 0)
    pltpu.sync_copy(dinvstr, shF.at[0, pl.ds(t * ST, ST)])
    plsc.subcore_barrier()
    pltpu.sync_copy(shF.at[0], segB)
    plsc.subcore_barrier()

    _fill(uA, NP, BIG, jnp.int32)
    _fill(uB, NP, BIG, jnp.int32)

    def phaseC(i, _):
        hv = idx_h[pl.ds(i * 16, 16)]
        nv = idx_n[pl.ds(i * 16, 16)]
        sv = plsc.load_gather(s_tab, [nv])
        mx = plsc.load_gather(segA, [hv])
        mn = plsc.load_gather(segB, [hv])
        cand_hi = jnp.where(sv == mx, nv, BIG)
        cand_lo = jnp.where(sv == mn, nv, BIG)
        _winner_rmw(conflict, hv, full, lane,
                    [(uA, cand_hi, jnp.minimum), (uB, cand_lo, jnp.minimum)])
        return 0

    for half in range(2):
        pltpu.sync_copy(nidx_hbm.at[pl.ds(t * EPT + half * EPH, EPH)], idx_n)
        pltpu.sync_copy(hidx_hbm.at[pl.ds(t * EPT + half * EPH, EPH)], idx_h)
        lax.fori_loop(0, EPH // 16, phaseC, 0)

    pltpu.sync_copy(uA, shI.at[t])
    plsc.subcore_barrier()
    body = _combine_stripe(shI, sti, t, jnp.minimum, BIG)(ustrA)
    lax.fori_loop(0, ST // 16, body, 0)
    plsc.subcore_barrier()
    pltpu.sync_copy(uB, shI.at[t])
    plsc.subcore_barrier()
    body = _combine_stripe(shI, sti, t, jnp.minimum, BIG)(ustrB)
    lax.fori_loop(0, ST // 16, body, 0)
    plsc.subcore_barrier()

    def phaseD1(j, _):
        ua = ustrA[pl.ds(j * 16, 16)]
        ub = ustrB[pl.ds(j * 16, 16)]
        valid = (ua < BIG) & (ub < BIG) & (ua != ub)
        srcb[pl.ds(j * 16, 16)] = jnp.where(valid, ua, 0)
        dstb[pl.ds(j * 16, 16)] = jnp.where(valid, ub, 0)
        return 0

    lax.fori_loop(0, ST // 16, phaseD1, 0)

    _fill(segA, NP, jnp.float32(0.0), jnp.float32)
    onef = jnp.full((16,), 1.0, jnp.float32)

    def phaseD2(j, _):
        ua = ustrA[pl.ds(j * 16, 16)]
        ub = ustrB[pl.ds(j * 16, 16)]
        sv16 = srcb[pl.ds(j * 16, 16)]
        dv16 = dstb[pl.ds(j * 16, 16)]
        valid = (ua < BIG) & (ub < BIG) & (ua != ub)
        _winner_rmw(conflict, sv16, valid, lane,
                    [(segA, onef, lambda c, v: c + v)])
        _winner_rmw(conflict, dv16, valid, lane,
                    [(segA, onef, lambda c, v: c + v)])
        return 0

    lax.fori_loop(0, ST // 16, phaseD2, 0)

    pltpu.sync_copy(segA, shF.at[t])
    plsc.subcore_barrier()

    def degbody(j, _):
        acc = jnp.full((16,), 0.0, jnp.float32)
        for k in range(16):
            acc = acc + stf[k, pl.ds(j * 16, 16)]
        deg = acc + 1.0
        bits = plsc.bitcast(deg, jnp.int32)
        y = plsc.bitcast(jnp.int32(0x5F3759DF) - (bits >> 1), jnp.float32)
        for _i in range(3):
            y = y * (1.5 - 0.5 * deg * y * y)
        dinvstr[pl.ds(j * 16, 16)] = y
        return 0

    lax.fori_loop(0, ST // 16, degbody, 0)
    pltpu.sync_copy(dinvstr, shF.at[0, pl.ds(t * ST, ST)])
    plsc.subcore_barrier()
    pltpu.sync_copy(shF.at[0], s_tab)
    plsc.subcore_barrier()

    def dsbody(j, _):
        y = dinvstr[pl.ds(j * 16, 16)]
        wbuf[pl.ds(j * 16, 16)] = y * y
        return 0

    lax.fori_loop(0, ST // 16, dsbody, 0)
    pltpu.sync_copy(wbuf.at[pl.ds(0, ST)], dscale_hbm.at[pl.ds(t * ST, ST)])

    def wbody(j, _):
        ua = ustrA[pl.ds(j * 16, 16)]
        ub = ustrB[pl.ds(j * 16, 16)]
        sv16 = srcb[pl.ds(j * 16, 16)]
        dv16 = dstb[pl.ds(j * 16, 16)]
        valid = (ua < BIG) & (ub < BIG) & (ua != ub)
        ds_ = plsc.load_gather(s_tab, [sv16])
        dd_ = plsc.load_gather(s_tab, [dv16])
        wv = jnp.where(valid, ds_ * dd_, 0.0)
        wbuf[pl.ds(j * 16, 16)] = wv
        wbuf[pl.ds(ST + j * 16, 16)] = wv
        return 0

    lax.fori_loop(0, ST // 16, wbody, 0)

    pltpu.sync_copy(srcb, ro_hbm.at[pl.ds(t * UPT, ST)])
    pltpu.sync_copy(dstb, ro_hbm.at[pl.ds(t * UPT + ST, ST)])
    pltpu.sync_copy(dstb, ri_hbm.at[pl.ds(t * UPT, ST)])
    pltpu.sync_copy(srcb, ri_hbm.at[pl.ds(t * UPT + ST, ST)])
    pltpu.sync_copy(wbuf, w_hbm.at[pl.ds(t * UPT, UPT)])


def _gb_kernel_body(s_hbm, nidx_hbm, hidx_hbm,
                    ro_hbm, ri_hbm, w_hbm, dscale_hbm, *scratch):
    c = lax.axis_index("c")
    pl.when(c == 0)(lambda: _gb_body(
        s_hbm, nidx_hbm, hidx_hbm, ro_hbm, ri_hbm, w_hbm, dscale_hbm,
        *scratch))


_graph_build = functools.partial(
    pl.kernel,
    out_type=[
        jax.ShapeDtypeStruct((NUPD,), jnp.int32),
        jax.ShapeDtypeStruct((NUPD,), jnp.int32),
        jax.ShapeDtypeStruct((NUPD,), jnp.float32),
        jax.ShapeDtypeStruct((NP,), jnp.float32),
    ],
    mesh=_mesh,
    compiler_params=pltpu.CompilerParams(needs_layout_passes=False),
    scratch_types=[
        pltpu.VMEM((NP,), jnp.float32),
        pltpu.VMEM((EPH,), jnp.int32),
        pltpu.VMEM((EPH,), jnp.int32),
        pltpu.VMEM((NP,), jnp.float32),
        pltpu.VMEM((NP,), jnp.float32),
        pltpu.VMEM((NP,), jnp.int32),
        pltpu.VMEM((NP,), jnp.int32),
        pltpu.VMEM((NP,), jnp.int32),
        pltpu.VMEM((16, ST), jnp.float32),
        pltpu.VMEM((16, ST), jnp.int32),
        pltpu.VMEM((ST,), jnp.int32),
        pltpu.VMEM((ST,), jnp.int32),
        pltpu.VMEM((ST,), jnp.int32),
        pltpu.VMEM((ST,), jnp.int32),
        pltpu.VMEM((ST,), jnp.float32),
        pltpu.VMEM((UPT,), jnp.float32),
        pltpu.VMEM_SHARED((16, NP), jnp.float32),
        pltpu.VMEM_SHARED((16, NP), jnp.int32),
    ],
)(_gb_kernel_body)


def _spmm_work(h_hbm, hinit_hbm, o_hbm, ro_hbm, ri_hbm, w_hbm,
               gbuf, rov, riv, wv, acc, semg, sems, t, f2):
    stripe = pl.ds(t * ST, ST)
    pltpu.sync_copy(hinit_hbm.at[stripe], acc.at[stripe])
    pltpu.sync_copy(ro_hbm.at[t], rov)
    pltpu.sync_copy(ri_hbm.at[t], riv)
    pltpu.sync_copy(w_hbm.at[pl.ds(t * UPT, UPT)], wv)
    plsc.subcore_barrier()

    gathers = [
        pltpu.async_copy(h_hbm.at[riv.at[chunk]],
                         gbuf.at[pl.ds(chunk * CH, CH)], semg)
        for chunk in range(NCH)
    ]
    for g in gathers:
        g.wait()

    def scale16(jj, _):
        w16 = wv[pl.ds(jj * 16, 16)]
        for k16 in range(16):
            sc = w16[k16]
            row = jj * 16 + k16
            for cc in range(f2 // 16):
                col = pl.ds(cc * 16, 16)
                gbuf[row, col] = gbuf[row, col] * sc
        return 0

    lax.fori_loop(0, UPT // 16, scale16, 0)

    scatters = [
        pltpu.async_copy(gbuf.at[pl.ds(chunk * CH, CH)],
                         acc.at[rov.at[chunk]], sems, add=True)
        for chunk in range(NCH)
    ]
    for sctr in scatters:
        sctr.wait()

    plsc.subcore_barrier()
    pltpu.sync_copy(acc.at[stripe], o_hbm.at[stripe])


def _make_spmm(f2):
    def body(ha, hb, hia, hib, ro3, ri3, w, oa, ob,
             gbuf, rov, riv, wv, acc, semg, sems):
        c = lax.axis_index("c")
        t = lax.axis_index("s")
        pl.when(c == 0)(lambda: _spmm_work(
            ha, hia, oa, ro3, ri3, w, gbuf, rov, riv, wv, acc, semg, sems,
            t, f2))
        pl.when(c == 1)(lambda: _spmm_work(
            hb, hib, ob, ro3, ri3, w, gbuf, rov, riv, wv, acc, semg, sems,
            t, f2))

    return functools.partial(
        pl.kernel,
        out_type=[
            jax.ShapeDtypeStruct((NP, f2), jnp.float32),
            jax.ShapeDtypeStruct((NP, f2), jnp.float32),
        ],
        mesh=_mesh,
        compiler_params=pltpu.CompilerParams(needs_layout_passes=False,
                                             use_tc_tiling_on_sc=False),
        scratch_types=[
            pltpu.VMEM((UPT, f2), jnp.float32),
            pltpu.VMEM((NCH, CH), jnp.int32),
            pltpu.VMEM((NCH, CH), jnp.int32),
            pltpu.VMEM((UPT,), jnp.float32),
            pltpu.VMEM_SHARED((NP, f2), jnp.float32),
            pltpu.SemaphoreType.DMA,
            pltpu.SemaphoreType.DMA,
        ],
    )(body)


_spmm64 = _make_spmm(64)
_spmm32 = _make_spmm(32)


def _mm_scale_kernel(x_ref, w_ref, b_ref, d_ref, ha, hb, hia, hib):
    h = jnp.dot(x_ref[...], w_ref[...],
                preferred_element_type=jnp.float32) + b_ref[...]
    hi = h * d_ref[...]
    half = h.shape[1] // 2
    ha[...] = h[:, :half]
    hb[...] = h[:, half:]
    hia[...] = hi[:, :half]
    hib[...] = hi[:, half:]


def _mm1(xp, W1, b1, dsc):
    blk = 2048
    m = W1.shape[1]
    half = m // 2
    sds = jax.ShapeDtypeStruct((NP, half), jnp.float32)
    return pl.pallas_call(
        _mm_scale_kernel,
        grid=(NP // blk,),
        in_specs=[
            pl.BlockSpec((blk, 128), lambda i: (i, 0)),
            pl.BlockSpec((128, m), lambda i: (0, 0)),
            pl.BlockSpec((m,), lambda i: (0,)),
            pl.BlockSpec((blk, 1), lambda i: (i, 0)),
        ],
        out_specs=[pl.BlockSpec((blk, half), lambda i: (i, 0))] * 4,
        out_shape=[sds, sds, sds, sds],
    )(xp, W1, b1, dsc[:, None])


def _mm2_kernel(a_ref, b_ref, w_ref, bias_ref, d_ref, oa, ob, oia, oib):
    h = jnp.concatenate([a_ref[...], b_ref[...]], axis=1)
    h = jax.nn.relu(h)
    o = jnp.dot(h, w_ref[...], preferred_element_type=jnp.float32) + bias_ref[...]
    oi = o * d_ref[...]
    half = o.shape[1] // 2
    oa[...] = o[:, :half]
    ob[...] = o[:, half:]
    oia[...] = oi[:, :half]
    oib[...] = oi[:, half:]


def _mm2(h1a, h1b, W2, b2, dsc):
    blk = 2048
    m = W2.shape[1]
    half = m // 2
    sds = jax.ShapeDtypeStruct((NP, half), jnp.float32)
    return pl.pallas_call(
        _mm2_kernel,
        grid=(NP // blk,),
        in_specs=[
            pl.BlockSpec((blk, 64), lambda i: (i, 0)),
            pl.BlockSpec((blk, 64), lambda i: (i, 0)),
            pl.BlockSpec((128, m), lambda i: (0, 0)),
            pl.BlockSpec((m,), lambda i: (0,)),
            pl.BlockSpec((blk, 1), lambda i: (i, 0)),
        ],
        out_specs=[pl.BlockSpec((blk, half), lambda i: (i, 0))] * 4,
        out_shape=[sds, sds, sds, sds],
    )(h1a, h1b, W2, b2, dsc[:, None])


def kernel(x, hyperedge_index, r, W1, b1, W2, b2):
    node_idx = hyperedge_index[0]
    he_idx = hyperedge_index[1]
    s = _matvec(x, r)
    s_pad = jnp.pad(s, (0, NP - N_NODES_C))
    ro, ri, w, dsc = _graph_build(s_pad, node_idx, he_idx)
    ro3 = ro.reshape(16, NCH, CH)
    ri3 = ri.reshape(16, NCH, CH)

    xp = jnp.pad(x, ((0, NP - N_NODES_C), (0, 0)))
    ha, hb, hia, hib = _mm1(xp, W1, b1, dsc)
    o1a, o1b = _spmm64(ha, hb, hia, hib, ro3, ri3, w)
    oa, ob, oia, oib = _mm2(o1a, o1b, W2, b2, dsc)
    qa, qb = _spmm32(oa, ob, oia, oib, ro3, ri3, w)
    return jnp.concatenate([qa[:N_NODES_C], qb[:N_NODES_C]], axis=1)


# speedup vs baseline: 26.5284x; 1.1196x over previous
import functools

import jax
import jax.numpy as jnp
from jax import lax
from jax.experimental import pallas as pl
from jax.experimental.pallas import tpu as pltpu
from jax.experimental.pallas import tpu_sc as plsc

N_NODES_C = 10000
N_HE_C = 10000
NNZ_C = 320000
NP = 10240
ST = 640
EPH2 = NNZ_C // 32
UPT = 2 * ST
NUPD = 16 * UPT
CH = 128
NCH = UPT // CH
BIG = N_NODES_C
NEGF = -3.0e38
POSF = 3.0e38

_mesh = plsc.VectorSubcoreMesh(core_axis_name="c", subcore_axis_name="s")


def _mm_kernel(x_ref, w_ref, b_ref, o_ref):
    o_ref[...] = jnp.dot(x_ref[...], w_ref[...],
                         preferred_element_type=jnp.float32) + b_ref[...]


def _matmul_bias(x, w, b):
    n, k = x.shape
    m = w.shape[1]
    blk = 2000
    return pl.pallas_call(
        _mm_kernel,
        grid=(n // blk,),
        in_specs=[
            pl.BlockSpec((blk, k), lambda i: (i, 0)),
            pl.BlockSpec((k, m), lambda i: (0, 0)),
            pl.BlockSpec((m,), lambda i: (0,)),
        ],
        out_specs=pl.BlockSpec((blk, m), lambda i: (i, 0)),
        out_shape=jax.ShapeDtypeStruct((n, m), jnp.float32),
    )(x, w, b)


def _matvec_kernel(x_ref, r_ref, o_ref):
    o_ref[...] = jnp.dot(x_ref[...], r_ref[...],
                         preferred_element_type=jnp.float32)


def _matvec(x, r):
    n, k = x.shape
    blk = 2000
    return pl.pallas_call(
        _matvec_kernel,
        grid=(n // blk,),
        in_specs=[
            pl.BlockSpec((blk, k), lambda i: (i, 0)),
            pl.BlockSpec((k, 128), lambda i: (0, 0)),
        ],
        out_specs=pl.BlockSpec((blk, 128), lambda i: (i, 0)),
        out_shape=jax.ShapeDtypeStruct((n, 128), jnp.float32),
    )(x, jnp.tile(r[:, None], (1, 128)))[:, 0]


def _fill(ref, nwords, val, dtype):
    vec = jnp.full((16,), val, dtype)

    def body(i, _):
        ref[pl.ds(i * 16, 16)] = vec
        return 0

    lax.fori_loop(0, nwords // 16, body, 0)


def _winner_rmw(conflict_ref, idx, mask0, lane, updates):
    def cond(pend):
        return jnp.any(pend)

    def body(pend):
        plsc.store_scatter(conflict_ref, [idx], lane, mask=pend)
        win = plsc.load_gather(conflict_ref, [idx], mask=pend)
        wm = pend & (win == lane)
        for ref, val, comb in updates:
            cur = plsc.load_gather(ref, [idx], mask=wm)
            plsc.store_scatter(ref, [idx], comb(cur, val), mask=wm)
        return pend & jnp.logical_not(wm)

    lax.while_loop(cond, body, mask0)


def _combine_stripe(sh, stf, t, op, init, nrefs=16):
    for k in range(nrefs):
        pltpu.sync_copy(sh.at[k, pl.ds(t * ST, ST)], stf.at[k])

    def make_body(out_ref):
        def body(j, _):
            acc = jnp.full((16,), init)
            for k in range(nrefs):
                acc = op(acc, stf[k, pl.ds(j * 16, 16)])
            out_ref[pl.ds(j * 16, 16)] = acc
            return 0
        return body

    return make_body


def _gba_work(s_hbm, nidx_hbm, hidx_hbm, mxP_hbm, mnP_hbm,
              s_tab, idx_n, idx_h, segA, segB, conflict, stf, strb, shF,
              c, t):
    g = c * 16 + t
    lane = lax.iota(jnp.int32, 16)
    full = jnp.full((16,), True)

    pltpu.sync_copy(s_hbm, s_tab)
    pltpu.sync_copy(nidx_hbm.at[pl.ds(g * EPH2, EPH2)], idx_n)
    pltpu.sync_copy(hidx_hbm.at[pl.ds(g * EPH2, EPH2)], idx_h)
    _fill(segA, NP, NEGF, jnp.float32)
    _fill(segB, NP, POSF, jnp.float32)

    def phaseB(i, _):
        hv = idx_h[pl.ds(i * 16, 16)]
        nv = idx_n[pl.ds(i * 16, 16)]
        sv = plsc.load_gather(s_tab, [nv])
        _winner_rmw(conflict, hv, full, lane,
                    [(segA, sv, jnp.maximum), (segB, sv, jnp.minimum)])
        return 0

    lax.fori_loop(0, EPH2 // 16, phaseB, 0)

    pltpu.sync_copy(segA, shF.at[t])
    plsc.subcore_barrier()
    body = _combine_stripe(shF, stf, t, jnp.maximum, NEGF)(strb)
    lax.fori_loop(0, ST // 16, body, 0)
    pltpu.sync_copy(strb, mxP_hbm.at[c, pl.ds(t * ST, ST)])
    plsc.subcore_barrier()
    pltpu.sync_copy(segB, shF.at[t])
    plsc.subcore_barrier()
    body = _combine_stripe(shF, stf, t, jnp.minimum, POSF)(strb)
    lax.fori_loop(0, ST // 16, body, 0)
    pltpu.sync_copy(strb, mnP_hbm.at[c, pl.ds(t * ST, ST)])


def _gba_body(s_hbm, nidx_hbm, hidx_hbm, mxP_hbm, mnP_hbm, *scratch):
    c = lax.axis_index("c")
    t = lax.axis_index("s")
    _gba_work(s_hbm, nidx_hbm, hidx_hbm, mxP_hbm, mnP_hbm, *scratch, c, t)


_gb_a = functools.partial(
    pl.kernel,
    out_type=[
        jax.ShapeDtypeStruct((2, NP), jnp.float32),
        jax.ShapeDtypeStruct((2, NP), jnp.float32),
    ],
    mesh=_mesh,
    compiler_params=pltpu.CompilerParams(needs_layout_passes=False),
    scratch_types=[
        pltpu.VMEM((NP,), jnp.float32),
        pltpu.VMEM((EPH2,), jnp.int32),
        pltpu.VMEM((EPH2,), jnp.int32),
        pltpu.VMEM((NP,), jnp.float32),
        pltpu.VMEM((NP,), jnp.float32),
        pltpu.VMEM((NP,), jnp.int32),
        pltpu.VMEM((16, ST), jnp.float32),
        pltpu.VMEM((ST,), jnp.float32),
        pltpu.VMEM_SHARED((16, NP), jnp.float32),
    ],
)(_gba_body)


def _elemwise2(dst, other, n, op):
    def body(i, _):
        sl = pl.ds(i * 16, 16)
        dst[sl] = op(dst[sl], other[sl])
        return 0

    lax.fori_loop(0, n // 16, body, 0)


def _gbb_work(s_hbm, nidx_hbm, hidx_hbm, mxP_hbm, mnP_hbm, uAP_hbm, uBP_hbm,
              s_tab, idx_n, idx_h, segA, segB, tmp, uA, uB, conflict,
              sti, strb, shI, c, t):
    g = c * 16 + t
    lane = lax.iota(jnp.int32, 16)
    full = jnp.full((16,), True)

    pltpu.sync_copy(s_hbm, s_tab)
    pltpu.sync_copy(nidx_hbm.at[pl.ds(g * EPH2, EPH2)], idx_n)
    pltpu.sync_copy(hidx_hbm.at[pl.ds(g * EPH2, EPH2)], idx_h)
    pltpu.sync_copy(mxP_hbm.at[0], segA)
    pltpu.sync_copy(mxP_hbm.at[1], tmp)
    _elemwise2(segA, tmp, NP, jnp.maximum)
    pltpu.sync_copy(mnP_hbm.at[0], segB)
    pltpu.sync_copy(mnP_hbm.at[1], tmp)
    _elemwise2(segB, tmp, NP, jnp.minimum)
    _fill(uA, NP, BIG, jnp.int32)
    _fill(uB, NP, BIG, jnp.int32)

    def phaseC(i, _):
        hv = idx_h[pl.ds(i * 16, 16)]
        nv = idx_n[pl.ds(i * 16, 16)]
        sv = plsc.load_gather(s_tab, [nv])
        mx = plsc.load_gather(segA, [hv])
        mn = plsc.load_gather(segB, [hv])
        cand_hi = jnp.where(sv == mx, nv, BIG)
        cand_lo = jnp.where(sv == mn, nv, BIG)
        _winner_rmw(conflict, hv, full, lane,
                    [(uA, cand_hi, jnp.minimum), (uB, cand_lo, jnp.minimum)])
        return 0

    lax.fori_loop(0, EPH2 // 16, phaseC, 0)

    pltpu.sync_copy(uA, shI.at[t])
    plsc.subcore_barrier()
    body = _combine_stripe(shI, sti, t, jnp.minimum, BIG)(strb)
    lax.fori_loop(0, ST // 16, body, 0)
    pltpu.sync_copy(strb, uAP_hbm.at[c, pl.ds(t * ST, ST)])
    plsc.subcore_barrier()
    pltpu.sync_copy(uB, shI.at[t])
    plsc.subcore_barrier()
    body = _combine_stripe(shI, sti, t, jnp.minimum, BIG)(strb)
    lax.fori_loop(0, ST // 16, body, 0)
    pltpu.sync_copy(strb, uBP_hbm.at[c, pl.ds(t * ST, ST)])


def _gbb_body(s_hbm, nidx_hbm, hidx_hbm, mxP_hbm, mnP_hbm,
              uAP_hbm, uBP_hbm, *scratch):
    c = lax.axis_index("c")
    t = lax.axis_index("s")
    _gbb_work(s_hbm, nidx_hbm, hidx_hbm, mxP_hbm, mnP_hbm, uAP_hbm, uBP_hbm,
              *scratch, c, t)


_gb_b = functools.partial(
    pl.kernel,
    out_type=[
        jax.ShapeDtypeStruct((2, NP), jnp.int32),
        jax.ShapeDtypeStruct((2, NP), jnp.int32),
    ],
    mesh=_mesh,
    compiler_params=pltpu.CompilerParams(needs_layout_passes=False),
    scratch_types=[
        pltpu.VMEM((NP,), jnp.float32),
        pltpu.VMEM((EPH2,), jnp.int32),
        pltpu.VMEM((EPH2,), jnp.int32),
        pltpu.VMEM((NP,), jnp.float32),
        pltpu.VMEM((NP,), jnp.float32),
        pltpu.VMEM((NP,), jnp.float32),
        pltpu.VMEM((NP,), jnp.int32),
        pltpu.VMEM((NP,), jnp.int32),
        pltpu.VMEM((NP,), jnp.int32),
        pltpu.VMEM((16, ST), jnp.int32),
        pltpu.VMEM((ST,), jnp.int32),
        pltpu.VMEM_SHARED((16, NP), jnp.int32),
    ],
)(_gbb_body)


def _gbc_work(uAP_hbm, uBP_hbm, ro_hbm, ri_hbm, w_hbm, dscale_hbm,
              s_tab, segA, conflict, stf, ustrA, ustrB, u2,
              srcb, dstb, dinvstr, wbuf, shF):
    t = lax.axis_index("s")
    lane = lax.iota(jnp.int32, 16)

    stripe = pl.ds(t * ST, ST)
    pltpu.sync_copy(uAP_hbm.at[0, stripe], ustrA)
    pltpu.sync_copy(uAP_hbm.at[1, stripe], u2)
    _elemwise2(ustrA, u2, ST, jnp.minimum)
    pltpu.sync_copy(uBP_hbm.at[0, stripe], ustrB)
    pltpu.sync_copy(uBP_hbm.at[1, stripe], u2)
    _elemwise2(ustrB, u2, ST, jnp.minimum)

    def phaseD1(j, _):
        ua = ustrA[pl.ds(j * 16, 16)]
        ub = ustrB[pl.ds(j * 16, 16)]
        valid = (ua < BIG) & (ub < BIG) & (ua != ub)
        srcb[pl.ds(j * 16, 16)] = jnp.where(valid, ua, 0)
        dstb[pl.ds(j * 16, 16)] = jnp.where(valid, ub, 0)
        return 0

    lax.fori_loop(0, ST // 16, phaseD1, 0)

    _fill(segA, NP, jnp.float32(0.0), jnp.float32)
    onef = jnp.full((16,), 1.0, jnp.float32)

    def phaseD2(j, _):
        ua = ustrA[pl.ds(j * 16, 16)]
        ub = ustrB[pl.ds(j * 16, 16)]
        sv16 = srcb[pl.ds(j * 16, 16)]
        dv16 = dstb[pl.ds(j * 16, 16)]
        valid = (ua < BIG) & (ub < BIG) & (ua != ub)
        _winner_rmw(conflict, sv16, valid, lane,
                    [(segA, onef, lambda c, v: c + v)])
        _winner_rmw(conflict, dv16, valid, lane,
                    [(segA, onef, lambda c, v: c + v)])
        return 0

    lax.fori_loop(0, ST // 16, phaseD2, 0)

    pltpu.sync_copy(segA, shF.at[t])
    plsc.subcore_barrier()

    def degbody(j, _):
        acc = jnp.full((16,), 0.0, jnp.float32)
        for k in range(16):
            acc = acc + stf[k, pl.ds(j * 16, 16)]
        deg = acc + 1.0
        bits = plsc.bitcast(deg, jnp.int32)
        y = plsc.bitcast(jnp.int32(0x5F3759DF) - (bits >> 1), jnp.float32)
        for _i in range(3):
            y = y * (1.5 - 0.5 * deg * y * y)
        dinvstr[pl.ds(j * 16, 16)] = y
        return 0

    lax.fori_loop(0, ST // 16, degbody, 0)
    pltpu.sync_copy(dinvstr, shF.at[0, pl.ds(t * ST, ST)])
    plsc.subcore_barrier()
    pltpu.sync_copy(shF.at[0], s_tab)
    plsc.subcore_barrier()

    def dsbody(j, _):
        y = dinvstr[pl.ds(j * 16, 16)]
        wbuf[pl.ds(j * 16, 16)] = y * y
        return 0

    lax.fori_loop(0, ST // 16, dsbody, 0)
    pltpu.sync_copy(wbuf.at[pl.ds(0, ST)], dscale_hbm.at[pl.ds(t * ST, ST)])

    def wbody(j, _):
        ua = ustrA[pl.ds(j * 16, 16)]
        ub = ustrB[pl.ds(j * 16, 16)]
        sv16 = srcb[pl.ds(j * 16, 16)]
        dv16 = dstb[pl.ds(j * 16, 16)]
        valid = (ua < BIG) & (ub < BIG) & (ua != ub)
        ds_ = plsc.load_gather(s_tab, [sv16])
        dd_ = plsc.load_gather(s_tab, [dv16])
        wv = jnp.where(valid, ds_ * dd_, 0.0)
        wbuf[pl.ds(j * 16, 16)] = wv
        wbuf[pl.ds(ST + j * 16, 16)] = wv
        return 0

    lax.fori_loop(0, ST // 16, wbody, 0)

    pltpu.sync_copy(srcb, ro_hbm.at[pl.ds(t * UPT, ST)])
    pltpu.sync_copy(dstb, ro_hbm.at[pl.ds(t * UPT + ST, ST)])
    pltpu.sync_copy(dstb, ri_hbm.at[pl.ds(t * UPT, ST)])
    pltpu.sync_copy(srcb, ri_hbm.at[pl.ds(t * UPT + ST, ST)])
    pltpu.sync_copy(wbuf, w_hbm.at[pl.ds(t * UPT, UPT)])


def _gbc_body(uAP_hbm, uBP_hbm, ro_hbm, ri_hbm, w_hbm, dscale_hbm, *scratch):
    c = lax.axis_index("c")
    pl.when(c == 0)(lambda: _gbc_work(
        uAP_hbm, uBP_hbm, ro_hbm, ri_hbm, w_hbm, dscale_hbm, *scratch))


_gb_c = functools.partial(
    pl.kernel,
    out_type=[
        jax.ShapeDtypeStruct((NUPD,), jnp.int32),
        jax.ShapeDtypeStruct((NUPD,), jnp.int32),
        jax.ShapeDtypeStruct((NUPD,), jnp.float32),
        jax.ShapeDtypeStruct((NP,), jnp.float32),
    ],
    mesh=_mesh,
    compiler_params=pltpu.CompilerParams(needs_layout_passes=False),
    scratch_types=[
        pltpu.VMEM((NP,), jnp.float32),
        pltpu.VMEM((NP,), jnp.float32),
        pltpu.VMEM((NP,), jnp.int32),
        pltpu.VMEM((16, ST), jnp.float32),
        pltpu.VMEM((ST,), jnp.int32),
        pltpu.VMEM((ST,), jnp.int32),
        pltpu.VMEM((ST,), jnp.int32),
        pltpu.VMEM((ST,), jnp.int32),
        pltpu.VMEM((ST,), jnp.int32),
        pltpu.VMEM((ST,), jnp.float32),
        pltpu.VMEM((UPT,), jnp.float32),
        pltpu.VMEM_SHARED((16, NP), jnp.float32),
    ],
)(_gbc_body)


def _spmm_work(h_hbm, hinit_hbm, o_hbm, ro_hbm, ri_hbm, w_hbm,
               gbuf, rov, riv, wv, acc, semg, sems, t, f2):
    stripe = pl.ds(t * ST, ST)
    pltpu.sync_copy(hinit_hbm.at[stripe], acc.at[stripe])
    pltpu.sync_copy(ro_hbm.at[t], rov)
    pltpu.sync_copy(ri_hbm.at[t], riv)
    pltpu.sync_copy(w_hbm.at[pl.ds(t * UPT, UPT)], wv)
    plsc.subcore_barrier()

    gathers = [
        pltpu.async_copy(h_hbm.at[riv.at[chunk]],
                         gbuf.at[pl.ds(chunk * CH, CH)], semg)
        for chunk in range(NCH)
    ]
    for g in gathers:
        g.wait()

    def scale16(jj, _):
        w16 = wv[pl.ds(jj * 16, 16)]
        for k16 in range(16):
            sc = w16[k16]
            row = jj * 16 + k16
            for cc in range(f2 // 16):
                col = pl.ds(cc * 16, 16)
                gbuf[row, col] = gbuf[row, col] * sc
        return 0

    lax.fori_loop(0, UPT // 16, scale16, 0)

    scatters = [
        pltpu.async_copy(gbuf.at[pl.ds(chunk * CH, CH)],
                         acc.at[rov.at[chunk]], sems, add=True)
        for chunk in range(NCH)
    ]
    for sctr in scatters:
        sctr.wait()

    plsc.subcore_barrier()
    pltpu.sync_copy(acc.at[stripe], o_hbm.at[stripe])


def _make_spmm(f2):
    def body(ha, hb, hia, hib, ro3, ri3, w, oa, ob,
             gbuf, rov, riv, wv, acc, semg, sems):
        c = lax.axis_index("c")
        t = lax.axis_index("s")
        pl.when(c == 0)(lambda: _spmm_work(
            ha, hia, oa, ro3, ri3, w, gbuf, rov, riv, wv, acc, semg, sems,
            t, f2))
        pl.when(c == 1)(lambda: _spmm_work(
            hb, hib, ob, ro3, ri3, w, gbuf, rov, riv, wv, acc, semg, sems,
            t, f2))

    return functools.partial(
        pl.kernel,
        out_type=[
            jax.ShapeDtypeStruct((NP, f2), jnp.float32),
            jax.ShapeDtypeStruct((NP, f2), jnp.float32),
        ],
        mesh=_mesh,
        compiler_params=pltpu.CompilerParams(needs_layout_passes=False,
                                             use_tc_tiling_on_sc=False),
        scratch_types=[
            pltpu.VMEM((UPT, f2), jnp.float32),
            pltpu.VMEM((NCH, CH), jnp.int32),
            pltpu.VMEM((NCH, CH), jnp.int32),
            pltpu.VMEM((UPT,), jnp.float32),
            pltpu.VMEM_SHARED((NP, f2), jnp.float32),
            pltpu.SemaphoreType.DMA,
            pltpu.SemaphoreType.DMA,
        ],
    )(body)


_spmm64 = _make_spmm(64)
_spmm32 = _make_spmm(32)


def _mm_scale_kernel(x_ref, w_ref, b_ref, d_ref, ha, hb, hia, hib):
    h = jnp.dot(x_ref[...], w_ref[...],
                preferred_element_type=jnp.float32) + b_ref[...]
    hi = h * d_ref[...]
    half = h.shape[1] // 2
    ha[...] = h[:, :half]
    hb[...] = h[:, half:]
    hia[...] = hi[:, :half]
    hib[...] = hi[:, half:]


def _mm1(xp, W1, b1, dsc):
    blk = 2048
    m = W1.shape[1]
    half = m // 2
    sds = jax.ShapeDtypeStruct((NP, half), jnp.float32)
    return pl.pallas_call(
        _mm_scale_kernel,
        grid=(NP // blk,),
        in_specs=[
            pl.BlockSpec((blk, 128), lambda i: (i, 0)),
            pl.BlockSpec((128, m), lambda i: (0, 0)),
            pl.BlockSpec((m,), lambda i: (0,)),
            pl.BlockSpec((blk, 1), lambda i: (i, 0)),
        ],
        out_specs=[pl.BlockSpec((blk, half), lambda i: (i, 0))] * 4,
        out_shape=[sds, sds, sds, sds],
    )(xp, W1, b1, dsc[:, None])


def _mm2_kernel(a_ref, b_ref, w_ref, bias_ref, d_ref, oa, ob, oia, oib):
    h = jnp.concatenate([a_ref[...], b_ref[...]], axis=1)
    h = jax.nn.relu(h)
    o = jnp.dot(h, w_ref[...], preferred_element_type=jnp.float32) + bias_ref[...]
    oi = o * d_ref[...]
    half = o.shape[1] // 2
    oa[...] = o[:, :half]
    ob[...] = o[:, half:]
    oia[...] = oi[:, :half]
    oib[...] = oi[:, half:]


def _mm2(h1a, h1b, W2, b2, dsc):
    blk = 2048
    m = W2.shape[1]
    half = m // 2
    sds = jax.ShapeDtypeStruct((NP, half), jnp.float32)
    return pl.pallas_call(
        _mm2_kernel,
        grid=(NP // blk,),
        in_specs=[
            pl.BlockSpec((blk, 64), lambda i: (i, 0)),
            pl.BlockSpec((blk, 64), lambda i: (i, 0)),
            pl.BlockSpec((128, m), lambda i: (0, 0)),
            pl.BlockSpec((m,), lambda i: (0,)),
            pl.BlockSpec((blk, 1), lambda i: (i, 0)),
        ],
        out_specs=[pl.BlockSpec((blk, half), lambda i: (i, 0))] * 4,
        out_shape=[sds, sds, sds, sds],
    )(h1a, h1b, W2, b2, dsc[:, None])


def kernel(x, hyperedge_index, r, W1, b1, W2, b2):
    node_idx = hyperedge_index[0]
    he_idx = hyperedge_index[1]
    s = _matvec(x, r)
    s_pad = jnp.pad(s, (0, NP - N_NODES_C))
    mxP, mnP = _gb_a(s_pad, node_idx, he_idx)
    uAP, uBP = _gb_b(s_pad, node_idx, he_idx, mxP, mnP)
    ro, ri, w, dsc = _gb_c(uAP, uBP)
    ro3 = ro.reshape(16, NCH, CH)
    ri3 = ri.reshape(16, NCH, CH)

    xp = jnp.pad(x, ((0, NP - N_NODES_C), (0, 0)))
    ha, hb, hia, hib = _mm1(xp, W1, b1, dsc)
    o1a, o1b = _spmm64(ha, hb, hia, hib, ro3, ri3, w)
    oa, ob, oia, oib = _mm2(o1a, o1b, W2, b2, dsc)
    qa, qb = _spmm32(oa, ob, oia, oib, ro3, ri3, w)
    return jnp.concatenate([qa[:N_NODES_C], qb[:N_NODES_C]], axis=1)


# speedup vs baseline: 27.2410x; 1.0269x over previous
import functools

import jax
import jax.numpy as jnp
from jax import lax
from jax.experimental import pallas as pl
from jax.experimental.pallas import tpu as pltpu
from jax.experimental.pallas import tpu_sc as plsc

N_NODES_C = 10000
N_HE_C = 10000
NNZ_C = 320000
NP = 10240
ST = 640
EPH2 = NNZ_C // 32
UPT = 2 * ST
NUPD = 16 * UPT
CH = 128
NCH = UPT // CH
BIG = N_NODES_C
NEGF = -3.0e38
POSF = 3.0e38

_mesh = plsc.VectorSubcoreMesh(core_axis_name="c", subcore_axis_name="s")


def _mm_kernel(x_ref, w_ref, b_ref, o_ref):
    o_ref[...] = jnp.dot(x_ref[...], w_ref[...],
                         preferred_element_type=jnp.float32) + b_ref[...]


def _matmul_bias(x, w, b):
    n, k = x.shape
    m = w.shape[1]
    blk = 2000
    return pl.pallas_call(
        _mm_kernel,
        grid=(n // blk,),
        in_specs=[
            pl.BlockSpec((blk, k), lambda i: (i, 0)),
            pl.BlockSpec((k, m), lambda i: (0, 0)),
            pl.BlockSpec((m,), lambda i: (0,)),
        ],
        out_specs=pl.BlockSpec((blk, m), lambda i: (i, 0)),
        out_shape=jax.ShapeDtypeStruct((n, m), jnp.float32),
    )(x, w, b)


def _matvec_kernel(x_ref, r_ref, o_ref):
    o_ref[...] = jnp.dot(x_ref[...], r_ref[...],
                         preferred_element_type=jnp.float32)


def _matvec(x, r):
    n, k = x.shape
    blk = 2000
    return pl.pallas_call(
        _matvec_kernel,
        grid=(n // blk,),
        in_specs=[
            pl.BlockSpec((blk, k), lambda i: (i, 0)),
            pl.BlockSpec((k, 128), lambda i: (0, 0)),
        ],
        out_specs=pl.BlockSpec((blk, 128), lambda i: (i, 0)),
        out_shape=jax.ShapeDtypeStruct((n, 128), jnp.float32),
    )(x, jnp.tile(r[:, None], (1, 128)))[:, 0]


def _fill(ref, nwords, val, dtype):
    vec = jnp.full((16,), val, dtype)

    def body(i, _):
        ref[pl.ds(i * 16, 16)] = vec
        return 0

    lax.fori_loop(0, nwords // 16, body, 0)


def _winner_rmw(conflict_ref, idx, mask0, lane, updates):
    def cond(pend):
        return jnp.any(pend)

    def body(pend):
        plsc.store_scatter(conflict_ref, [idx], lane, mask=pend)
        win = plsc.load_gather(conflict_ref, [idx], mask=pend)
        wm = pend & (win == lane)
        for ref, val, comb in updates:
            cur = plsc.load_gather(ref, [idx], mask=wm)
            plsc.store_scatter(ref, [idx], comb(cur, val), mask=wm)
        return pend & jnp.logical_not(wm)

    lax.while_loop(cond, body, mask0)


def _combine_stripe(sh, stf, t, op, init, nrefs=16):
    for k in range(nrefs):
        pltpu.sync_copy(sh.at[k, pl.ds(t * ST, ST)], stf.at[k])

    def make_body(out_ref):
        def body(j, _):
            acc = jnp.full((16,), init)
            for k in range(nrefs):
                acc = op(acc, stf[k, pl.ds(j * 16, 16)])
            out_ref[pl.ds(j * 16, 16)] = acc
            return 0
        return body

    return make_body


def _gba_work(s_hbm, nidx_hbm, hidx_hbm, mxP_hbm, mnP_hbm,
              s_tab, idx_n, idx_h, segA, segB, conflict, stf, strb, shF,
              c, t):
    g = c * 16 + t
    lane = lax.iota(jnp.int32, 16)
    full = jnp.full((16,), True)

    pltpu.sync_copy(s_hbm, s_tab)
    pltpu.sync_copy(nidx_hbm.at[pl.ds(g * EPH2, EPH2)], idx_n)
    pltpu.sync_copy(hidx_hbm.at[pl.ds(g * EPH2, EPH2)], idx_h)
    _fill(segA, NP, NEGF, jnp.float32)
    _fill(segB, NP, POSF, jnp.float32)

    def phaseB(i, _):
        hv = idx_h[pl.ds(i * 16, 16)]
        nv = idx_n[pl.ds(i * 16, 16)]
        sv = plsc.load_gather(s_tab, [nv])
        _winner_rmw(conflict, hv, full, lane,
                    [(segA, sv, jnp.maximum), (segB, sv, jnp.minimum)])
        return 0

    lax.fori_loop(0, EPH2 // 16, phaseB, 0, unroll=4)

    pltpu.sync_copy(segA, shF.at[t])
    plsc.subcore_barrier()
    body = _combine_stripe(shF, stf, t, jnp.maximum, NEGF)(strb)
    lax.fori_loop(0, ST // 16, body, 0)
    pltpu.sync_copy(strb, mxP_hbm.at[c, pl.ds(t * ST, ST)])
    plsc.subcore_barrier()
    pltpu.sync_copy(segB, shF.at[t])
    plsc.subcore_barrier()
    body = _combine_stripe(shF, stf, t, jnp.minimum, POSF)(strb)
    lax.fori_loop(0, ST // 16, body, 0)
    pltpu.sync_copy(strb, mnP_hbm.at[c, pl.ds(t * ST, ST)])


def _gba_body(s_hbm, nidx_hbm, hidx_hbm, mxP_hbm, mnP_hbm, *scratch):
    c = lax.axis_index("c")
    t = lax.axis_index("s")
    _gba_work(s_hbm, nidx_hbm, hidx_hbm, mxP_hbm, mnP_hbm, *scratch, c, t)


_gb_a = functools.partial(
    pl.kernel,
    out_type=[
        jax.ShapeDtypeStruct((2, NP), jnp.float32),
        jax.ShapeDtypeStruct((2, NP), jnp.float32),
    ],
    mesh=_mesh,
    compiler_params=pltpu.CompilerParams(needs_layout_passes=False),
    scratch_types=[
        pltpu.VMEM((NP,), jnp.float32),
        pltpu.VMEM((EPH2,), jnp.int32),
        pltpu.VMEM((EPH2,), jnp.int32),
        pltpu.VMEM((NP,), jnp.float32),
        pltpu.VMEM((NP,), jnp.float32),
        pltpu.VMEM((NP,), jnp.int32),
        pltpu.VMEM((16, ST), jnp.float32),
        pltpu.VMEM((ST,), jnp.float32),
        pltpu.VMEM_SHARED((16, NP), jnp.float32),
    ],
)(_gba_body)


def _elemwise2(dst, other, n, op):
    def body(i, _):
        sl = pl.ds(i * 16, 16)
        dst[sl] = op(dst[sl], other[sl])
        return 0

    lax.fori_loop(0, n // 16, body, 0)


def _gbb_work(s_hbm, nidx_hbm, hidx_hbm, mxP_hbm, mnP_hbm, uAP_hbm, uBP_hbm,
              s_tab, idx_n, idx_h, segA, segB, tmp, uA, uB, conflict,
              sti, strb, shI, c, t):
    g = c * 16 + t
    lane = lax.iota(jnp.int32, 16)
    full = jnp.full((16,), True)

    pltpu.sync_copy(s_hbm, s_tab)
    pltpu.sync_copy(nidx_hbm.at[pl.ds(g * EPH2, EPH2)], idx_n)
    pltpu.sync_copy(hidx_hbm.at[pl.ds(g * EPH2, EPH2)], idx_h)
    pltpu.sync_copy(mxP_hbm.at[0], segA)
    pltpu.sync_copy(mxP_hbm.at[1], tmp)
    _elemwise2(segA, tmp, NP, jnp.maximum)
    pltpu.sync_copy(mnP_hbm.at[0], segB)
    pltpu.sync_copy(mnP_hbm.at[1], tmp)
    _elemwise2(segB, tmp, NP, jnp.minimum)
    _fill(uA, NP, BIG, jnp.int32)
    _fill(uB, NP, BIG, jnp.int32)

    def phaseC(i, _):
        hv = idx_h[pl.ds(i * 16, 16)]
        nv = idx_n[pl.ds(i * 16, 16)]
        sv = plsc.load_gather(s_tab, [nv])
        mx = plsc.load_gather(segA, [hv])
        mn = plsc.load_gather(segB, [hv])
        cand_hi = jnp.where(sv == mx, nv, BIG)
        cand_lo = jnp.where(sv == mn, nv, BIG)
        _winner_rmw(conflict, hv, full, lane,
                    [(uA, cand_hi, jnp.minimum), (uB, cand_lo, jnp.minimum)])
        return 0

    lax.fori_loop(0, EPH2 // 16, phaseC, 0, unroll=4)

    pltpu.sync_copy(uA, shI.at[t])
    plsc.subcore_barrier()
    body = _combine_stripe(shI, sti, t, jnp.minimum, BIG)(strb)
    lax.fori_loop(0, ST // 16, body, 0)
    pltpu.sync_copy(strb, uAP_hbm.at[c, pl.ds(t * ST, ST)])
    plsc.subcore_barrier()
    pltpu.sync_copy(uB, shI.at[t])
    plsc.subcore_barrier()
    body = _combine_stripe(shI, sti, t, jnp.minimum, BIG)(strb)
    lax.fori_loop(0, ST // 16, body, 0)
    pltpu.sync_copy(strb, uBP_hbm.at[c, pl.ds(t * ST, ST)])


def _gbb_body(s_hbm, nidx_hbm, hidx_hbm, mxP_hbm, mnP_hbm,
              uAP_hbm, uBP_hbm, *scratch):
    c = lax.axis_index("c")
    t = lax.axis_index("s")
    _gbb_work(s_hbm, nidx_hbm, hidx_hbm, mxP_hbm, mnP_hbm, uAP_hbm, uBP_hbm,
              *scratch, c, t)


_gb_b = functools.partial(
    pl.kernel,
    out_type=[
        jax.ShapeDtypeStruct((2, NP), jnp.int32),
        jax.ShapeDtypeStruct((2, NP), jnp.int32),
    ],
    mesh=_mesh,
    compiler_params=pltpu.CompilerParams(needs_layout_passes=False),
    scratch_types=[
        pltpu.VMEM((NP,), jnp.float32),
        pltpu.VMEM((EPH2,), jnp.int32),
        pltpu.VMEM((EPH2,), jnp.int32),
        pltpu.VMEM((NP,), jnp.float32),
        pltpu.VMEM((NP,), jnp.float32),
        pltpu.VMEM((NP,), jnp.float32),
        pltpu.VMEM((NP,), jnp.int32),
        pltpu.VMEM((NP,), jnp.int32),
        pltpu.VMEM((NP,), jnp.int32),
        pltpu.VMEM((16, ST), jnp.int32),
        pltpu.VMEM((ST,), jnp.int32),
        pltpu.VMEM_SHARED((16, NP), jnp.int32),
    ],
)(_gbb_body)


def _gbc_work(uAP_hbm, uBP_hbm, ro_hbm, ri_hbm, w_hbm, dscale_hbm,
              s_tab, segA, conflict, stf, ustrA, ustrB, u2,
              srcb, dstb, dinvstr, wbuf, shF):
    t = lax.axis_index("s")
    lane = lax.iota(jnp.int32, 16)

    stripe = pl.ds(t * ST, ST)
    pltpu.sync_copy(uAP_hbm.at[0, stripe], ustrA)
    pltpu.sync_copy(uAP_hbm.at[1, stripe], u2)
    _elemwise2(ustrA, u2, ST, jnp.minimum)
    pltpu.sync_copy(uBP_hbm.at[0, stripe], ustrB)
    pltpu.sync_copy(uBP_hbm.at[1, stripe], u2)
    _elemwise2(ustrB, u2, ST, jnp.minimum)

    def phaseD1(j, _):
        ua = ustrA[pl.ds(j * 16, 16)]
        ub = ustrB[pl.ds(j * 16, 16)]
        valid = (ua < BIG) & (ub < BIG) & (ua != ub)
        srcb[pl.ds(j * 16, 16)] = jnp.where(valid, ua, 0)
        dstb[pl.ds(j * 16, 16)] = jnp.where(valid, ub, 0)
        return 0

    lax.fori_loop(0, ST // 16, phaseD1, 0)

    _fill(segA, NP, jnp.float32(0.0), jnp.float32)
    onef = jnp.full((16,), 1.0, jnp.float32)

    def phaseD2(j, _):
        ua = ustrA[pl.ds(j * 16, 16)]
        ub = ustrB[pl.ds(j * 16, 16)]
        sv16 = srcb[pl.ds(j * 16, 16)]
        dv16 = dstb[pl.ds(j * 16, 16)]
        valid = (ua < BIG) & (ub < BIG) & (ua != ub)
        _winner_rmw(conflict, sv16, valid, lane,
                    [(segA, onef, lambda c, v: c + v)])
        _winner_rmw(conflict, dv16, valid, lane,
                    [(segA, onef, lambda c, v: c + v)])
        return 0

    lax.fori_loop(0, ST // 16, phaseD2, 0)

    pltpu.sync_copy(segA, shF.at[t])
    plsc.subcore_barrier()

    def degbody(j, _):
        acc = jnp.full((16,), 0.0, jnp.float32)
        for k in range(16):
            acc = acc + stf[k, pl.ds(j * 16, 16)]
        deg = acc + 1.0
        bits = plsc.bitcast(deg, jnp.int32)
        y = plsc.bitcast(jnp.int32(0x5F3759DF) - (bits >> 1), jnp.float32)
        for _i in range(3):
            y = y * (1.5 - 0.5 * deg * y * y)
        dinvstr[pl.ds(j * 16, 16)] = y
        return 0

    lax.fori_loop(0, ST // 16, degbody, 0)
    pltpu.sync_copy(dinvstr, shF.at[0, pl.ds(t * ST, ST)])
    plsc.subcore_barrier()
    pltpu.sync_copy(shF.at[0], s_tab)
    plsc.subcore_barrier()

    def dsbody(j, _):
        y = dinvstr[pl.ds(j * 16, 16)]
        wbuf[pl.ds(j * 16, 16)] = y * y
        return 0

    lax.fori_loop(0, ST // 16, dsbody, 0)
    pltpu.sync_copy(wbuf.at[pl.ds(0, ST)], dscale_hbm.at[pl.ds(t * ST, ST)])

    def wbody(j, _):
        ua = ustrA[pl.ds(j * 16, 16)]
        ub = ustrB[pl.ds(j * 16, 16)]
        sv16 = srcb[pl.ds(j * 16, 16)]
        dv16 = dstb[pl.ds(j * 16, 16)]
        valid = (ua < BIG) & (ub < BIG) & (ua != ub)
        ds_ = plsc.load_gather(s_tab, [sv16])
        dd_ = plsc.load_gather(s_tab, [dv16])
        wv = jnp.where(valid, ds_ * dd_, 0.0)
        wbuf[pl.ds(j * 16, 16)] = wv
        wbuf[pl.ds(ST + j * 16, 16)] = wv
        return 0

    lax.fori_loop(0, ST // 16, wbody, 0)

    pltpu.sync_copy(srcb, ro_hbm.at[pl.ds(t * UPT, ST)])
    pltpu.sync_copy(dstb, ro_hbm.at[pl.ds(t * UPT + ST, ST)])
    pltpu.sync_copy(dstb, ri_hbm.at[pl.ds(t * UPT, ST)])
    pltpu.sync_copy(srcb, ri_hbm.at[pl.ds(t * UPT + ST, ST)])
    pltpu.sync_copy(wbuf, w_hbm.at[pl.ds(t * UPT, UPT)])


def _gbc_body(uAP_hbm, uBP_hbm, ro_hbm, ri_hbm, w_hbm, dscale_hbm, *scratch):
    c = lax.axis_index("c")
    pl.when(c == 0)(lambda: _gbc_work(
        uAP_hbm, uBP_hbm, ro_hbm, ri_hbm, w_hbm, dscale_hbm, *scratch))


_gb_c = functools.partial(
    pl.kernel,
    out_type=[
        jax.ShapeDtypeStruct((NUPD,), jnp.int32),
        jax.ShapeDtypeStruct((NUPD,), jnp.int32),
        jax.ShapeDtypeStruct((NUPD,), jnp.float32),
        jax.ShapeDtypeStruct((NP,), jnp.float32),
    ],
    mesh=_mesh,
    compiler_params=pltpu.CompilerParams(needs_layout_passes=False),
    scratch_types=[
        pltpu.VMEM((NP,), jnp.float32),
        pltpu.VMEM((NP,), jnp.float32),
        pltpu.VMEM((NP,), jnp.int32),
        pltpu.VMEM((16, ST), jnp.float32),
        pltpu.VMEM((ST,), jnp.int32),
        pltpu.VMEM((ST,), jnp.int32),
        pltpu.VMEM((ST,), jnp.int32),
        pltpu.VMEM((ST,), jnp.int32),
        pltpu.VMEM((ST,), jnp.int32),
        pltpu.VMEM((ST,), jnp.float32),
        pltpu.VMEM((UPT,), jnp.float32),
        pltpu.VMEM_SHARED((16, NP), jnp.float32),
    ],
)(_gbc_body)


def _spmm_work(h_hbm, hinit_hbm, o_hbm, ro_hbm, ri_hbm, w_hbm,
               gbuf, rov, riv, wv, acc, semg, sems, t, f2):
    stripe = pl.ds(t * ST, ST)
    pltpu.sync_copy(hinit_hbm.at[stripe], acc.at[stripe])
    pltpu.sync_copy(ro_hbm.at[t], rov)
    pltpu.sync_copy(ri_hbm.at[t], riv)
    pltpu.sync_copy(w_hbm.at[pl.ds(t * UPT, UPT)], wv)
    plsc.subcore_barrier()

    gathers = [
        pltpu.async_copy(h_hbm.at[riv.at[chunk]],
                         gbuf.at[pl.ds(chunk * CH, CH)], semg)
        for chunk in range(NCH)
    ]
    for g in gathers:
        g.wait()

    def scale16(jj, _):
        w16 = wv[pl.ds(jj * 16, 16)]
        for k16 in range(16):
            sc = w16[k16]
            row = jj * 16 + k16
            for cc in range(f2 // 16):
                col = pl.ds(cc * 16, 16)
                gbuf[row, col] = gbuf[row, col] * sc
        return 0

    lax.fori_loop(0, UPT // 16, scale16, 0)

    scatters = [
        pltpu.async_copy(gbuf.at[pl.ds(chunk * CH, CH)],
                         acc.at[rov.at[chunk]], sems, add=True)
        for chunk in range(NCH)
    ]
    for sctr in scatters:
        sctr.wait()

    plsc.subcore_barrier()
    pltpu.sync_copy(acc.at[stripe], o_hbm.at[stripe])


def _make_spmm(f2):
    def body(ha, hb, hia, hib, ro3, ri3, w, oa, ob,
             gbuf, rov, riv, wv, acc, semg, sems):
        c = lax.axis_index("c")
        t = lax.axis_index("s")
        pl.when(c == 0)(lambda: _spmm_work(
            ha, hia, oa, ro3, ri3, w, gbuf, rov, riv, wv, acc, semg, sems,
            t, f2))
        pl.when(c == 1)(lambda: _spmm_work(
            hb, hib, ob, ro3, ri3, w, gbuf, rov, riv, wv, acc, semg, sems,
            t, f2))

    return functools.partial(
        pl.kernel,
        out_type=[
            jax.ShapeDtypeStruct((NP, f2), jnp.float32),
            jax.ShapeDtypeStruct((NP, f2), jnp.float32),
        ],
        mesh=_mesh,
        compiler_params=pltpu.CompilerParams(needs_layout_passes=False,
                                             use_tc_tiling_on_sc=False),
        scratch_types=[
            pltpu.VMEM((UPT, f2), jnp.float32),
            pltpu.VMEM((NCH, CH), jnp.int32),
            pltpu.VMEM((NCH, CH), jnp.int32),
            pltpu.VMEM((UPT,), jnp.float32),
            pltpu.VMEM_SHARED((NP, f2), jnp.float32),
            pltpu.SemaphoreType.DMA,
            pltpu.SemaphoreType.DMA,
        ],
    )(body)


_spmm64 = _make_spmm(64)
_spmm32 = _make_spmm(32)


def _mm_scale_kernel(x_ref, w_ref, b_ref, d_ref, ha, hb, hia, hib):
    h = jnp.dot(x_ref[...], w_ref[...],
                preferred_element_type=jnp.float32) + b_ref[...]
    hi = h * d_ref[...]
    half = h.shape[1] // 2
    ha[...] = h[:, :half]
    hb[...] = h[:, half:]
    hia[...] = hi[:, :half]
    hib[...] = hi[:, half:]


def _mm1(xp, W1, b1, dsc):
    blk = 2048
    m = W1.shape[1]
    half = m // 2
    sds = jax.ShapeDtypeStruct((NP, half), jnp.float32)
    return pl.pallas_call(
        _mm_scale_kernel,
        grid=(NP // blk,),
        in_specs=[
            pl.BlockSpec((blk, 128), lambda i: (i, 0)),
            pl.BlockSpec((128, m), lambda i: (0, 0)),
            pl.BlockSpec((m,), lambda i: (0,)),
            pl.BlockSpec((blk, 1), lambda i: (i, 0)),
        ],
        out_specs=[pl.BlockSpec((blk, half), lambda i: (i, 0))] * 4,
        out_shape=[sds, sds, sds, sds],
    )(xp, W1, b1, dsc[:, None])


def _mm2_kernel(a_ref, b_ref, w_ref, bias_ref, d_ref, oa, ob, oia, oib):
    h = jnp.concatenate([a_ref[...], b_ref[...]], axis=1)
    h = jax.nn.relu(h)
    o = jnp.dot(h, w_ref[...], preferred_element_type=jnp.float32) + bias_ref[...]
    oi = o * d_ref[...]
    half = o.shape[1] // 2
    oa[...] = o[:, :half]
    ob[...] = o[:, half:]
    oia[...] = oi[:, :half]
    oib[...] = oi[:, half:]


def _mm2(h1a, h1b, W2, b2, dsc):
    blk = 2048
    m = W2.shape[1]
    half = m // 2
    sds = jax.ShapeDtypeStruct((NP, half), jnp.float32)
    return pl.pallas_call(
        _mm2_kernel,
        grid=(NP // blk,),
        in_specs=[
            pl.BlockSpec((blk, 64), lambda i: (i, 0)),
            pl.BlockSpec((blk, 64), lambda i: (i, 0)),
            pl.BlockSpec((128, m), lambda i: (0, 0)),
            pl.BlockSpec((m,), lambda i: (0,)),
            pl.BlockSpec((blk, 1), lambda i: (i, 0)),
        ],
        out_specs=[pl.BlockSpec((blk, half), lambda i: (i, 0))] * 4,
        out_shape=[sds, sds, sds, sds],
    )(h1a, h1b, W2, b2, dsc[:, None])


def kernel(x, hyperedge_index, r, W1, b1, W2, b2):
    node_idx = hyperedge_index[0]
    he_idx = hyperedge_index[1]
    s = _matvec(x, r)
    s_pad = jnp.pad(s, (0, NP - N_NODES_C))
    mxP, mnP = _gb_a(s_pad, node_idx, he_idx)
    uAP, uBP = _gb_b(s_pad, node_idx, he_idx, mxP, mnP)
    ro, ri, w, dsc = _gb_c(uAP, uBP)
    ro3 = ro.reshape(16, NCH, CH)
    ri3 = ri.reshape(16, NCH, CH)

    xp = jnp.pad(x, ((0, NP - N_NODES_C), (0, 0)))
    ha, hb, hia, hib = _mm1(xp, W1, b1, dsc)
    o1a, o1b = _spmm64(ha, hb, hia, hib, ro3, ri3, w)
    oa, ob, oia, oib = _mm2(o1a, o1b, W2, b2, dsc)
    qa, qb = _spmm32(oa, ob, oia, oib, ro3, ri3, w)
    return jnp.concatenate([qa[:N_NODES_C], qb[:N_NODES_C]], axis=1)


# speedup vs baseline: 28.1423x; 1.0331x over previous
import functools

import jax
import jax.numpy as jnp
from jax import lax
from jax.experimental import pallas as pl
from jax.experimental.pallas import tpu as pltpu
from jax.experimental.pallas import tpu_sc as plsc

N_NODES_C = 10000
N_HE_C = 10000
NNZ_C = 320000
NP = 10240
ST = 640
EPH2 = NNZ_C // 32
UPT = 3 * ST
NUPD = 16 * UPT
CH = 128
NCH = UPT // CH
NCH_P1 = 8
BIG = N_NODES_C
NEGF = -3.0e38
POSF = 3.0e38

_mesh = plsc.VectorSubcoreMesh(core_axis_name="c", subcore_axis_name="s")


def _mm_kernel(x_ref, w_ref, b_ref, o_ref):
    o_ref[...] = jnp.dot(x_ref[...], w_ref[...],
                         preferred_element_type=jnp.float32) + b_ref[...]


def _matmul_bias(x, w, b):
    n, k = x.shape
    m = w.shape[1]
    blk = 2000
    return pl.pallas_call(
        _mm_kernel,
        grid=(n // blk,),
        in_specs=[
            pl.BlockSpec((blk, k), lambda i: (i, 0)),
            pl.BlockSpec((k, m), lambda i: (0, 0)),
            pl.BlockSpec((m,), lambda i: (0,)),
        ],
        out_specs=pl.BlockSpec((blk, m), lambda i: (i, 0)),
        out_shape=jax.ShapeDtypeStruct((n, m), jnp.float32),
    )(x, w, b)


def _matvec_kernel(x_ref, r_ref, o_ref):
    o_ref[...] = jnp.dot(x_ref[...], r_ref[...],
                         preferred_element_type=jnp.float32)


def _matvec(x, r):
    n, k = x.shape
    blk = 2000
    return pl.pallas_call(
        _matvec_kernel,
        grid=(n // blk,),
        in_specs=[
            pl.BlockSpec((blk, k), lambda i: (i, 0)),
            pl.BlockSpec((k, 128), lambda i: (0, 0)),
        ],
        out_specs=pl.BlockSpec((blk, 128), lambda i: (i, 0)),
        out_shape=jax.ShapeDtypeStruct((n, 128), jnp.float32),
    )(x, jnp.tile(r[:, None], (1, 128)))[:, 0]


def _fill(ref, nwords, val, dtype):
    vec = jnp.full((16,), val, dtype)

    def body(i, _):
        ref[pl.ds(i * 16, 16)] = vec
        return 0

    lax.fori_loop(0, nwords // 16, body, 0)


def _winner_rmw(conflict_ref, idx, mask0, lane, updates):
    def cond(pend):
        return jnp.any(pend)

    def body(pend):
        plsc.store_scatter(conflict_ref, [idx], lane, mask=pend)
        win = plsc.load_gather(conflict_ref, [idx], mask=pend)
        wm = pend & (win == lane)
        for ref, val, comb in updates:
            cur = plsc.load_gather(ref, [idx], mask=wm)
            plsc.store_scatter(ref, [idx], comb(cur, val), mask=wm)
        return pend & jnp.logical_not(wm)

    lax.while_loop(cond, body, mask0)


def _combine_stripe(sh, stf, t, op, init, nrefs=16):
    for k in range(nrefs):
        pltpu.sync_copy(sh.at[k, pl.ds(t * ST, ST)], stf.at[k])

    def make_body(out_ref):
        def body(j, _):
            acc = jnp.full((16,), init)
            for k in range(nrefs):
                acc = op(acc, stf[k, pl.ds(j * 16, 16)])
            out_ref[pl.ds(j * 16, 16)] = acc
            return 0
        return body

    return make_body


def _gba_work(s_hbm, nidx_hbm, hidx_hbm, mxP_hbm, mnP_hbm,
              s_tab, idx_n, idx_h, segA, segB, conflict, stf, strb, shF,
              c, t):
    g = c * 16 + t
    lane = lax.iota(jnp.int32, 16)
    full = jnp.full((16,), True)

    pltpu.sync_copy(s_hbm, s_tab)
    pltpu.sync_copy(nidx_hbm.at[pl.ds(g * EPH2, EPH2)], idx_n)
    pltpu.sync_copy(hidx_hbm.at[pl.ds(g * EPH2, EPH2)], idx_h)
    _fill(segA, NP, NEGF, jnp.float32)
    _fill(segB, NP, POSF, jnp.float32)

    def phaseB(i, _):
        hv = idx_h[pl.ds(i * 16, 16)]
        nv = idx_n[pl.ds(i * 16, 16)]
        sv = plsc.load_gather(s_tab, [nv])
        _winner_rmw(conflict, hv, full, lane,
                    [(segA, sv, jnp.maximum), (segB, sv, jnp.minimum)])
        return 0

    lax.fori_loop(0, EPH2 // 16, phaseB, 0, unroll=4)

    pltpu.sync_copy(segA, shF.at[t])
    plsc.subcore_barrier()
    body = _combine_stripe(shF, stf, t, jnp.maximum, NEGF)(strb)
    lax.fori_loop(0, ST // 16, body, 0)
    pltpu.sync_copy(strb, mxP_hbm.at[c, pl.ds(t * ST, ST)])
    plsc.subcore_barrier()
    pltpu.sync_copy(segB, shF.at[t])
    plsc.subcore_barrier()
    body = _combine_stripe(shF, stf, t, jnp.minimum, POSF)(strb)
    lax.fori_loop(0, ST // 16, body, 0)
    pltpu.sync_copy(strb, mnP_hbm.at[c, pl.ds(t * ST, ST)])


def _gba_body(s_hbm, nidx_hbm, hidx_hbm, mxP_hbm, mnP_hbm, *scratch):
    c = lax.axis_index("c")
    t = lax.axis_index("s")
    _gba_work(s_hbm, nidx_hbm, hidx_hbm, mxP_hbm, mnP_hbm, *scratch, c, t)


_gb_a = functools.partial(
    pl.kernel,
    out_type=[
        jax.ShapeDtypeStruct((2, NP), jnp.float32),
        jax.ShapeDtypeStruct((2, NP), jnp.float32),
    ],
    mesh=_mesh,
    compiler_params=pltpu.CompilerParams(needs_layout_passes=False),
    scratch_types=[
        pltpu.VMEM((NP,), jnp.float32),
        pltpu.VMEM((EPH2,), jnp.int32),
        pltpu.VMEM((EPH2,), jnp.int32),
        pltpu.VMEM((NP,), jnp.float32),
        pltpu.VMEM((NP,), jnp.float32),
        pltpu.VMEM((NP,), jnp.int32),
        pltpu.VMEM((16, ST), jnp.float32),
        pltpu.VMEM((ST,), jnp.float32),
        pltpu.VMEM_SHARED((16, NP), jnp.float32),
    ],
)(_gba_body)


def _elemwise2(dst, other, n, op):
    def body(i, _):
        sl = pl.ds(i * 16, 16)
        dst[sl] = op(dst[sl], other[sl])
        return 0

    lax.fori_loop(0, n // 16, body, 0)


def _gbb_work(s_hbm, nidx_hbm, hidx_hbm, mxP_hbm, mnP_hbm, uAP_hbm, uBP_hbm,
              s_tab, idx_n, idx_h, segA, segB, tmp, uA, uB, conflict,
              sti, strb, shI, c, t):
    g = c * 16 + t
    lane = lax.iota(jnp.int32, 16)
    full = jnp.full((16,), True)

    pltpu.sync_copy(s_hbm, s_tab)
    pltpu.sync_copy(nidx_hbm.at[pl.ds(g * EPH2, EPH2)], idx_n)
    pltpu.sync_copy(hidx_hbm.at[pl.ds(g * EPH2, EPH2)], idx_h)
    pltpu.sync_copy(mxP_hbm.at[0], segA)
    pltpu.sync_copy(mxP_hbm.at[1], tmp)
    _elemwise2(segA, tmp, NP, jnp.maximum)
    pltpu.sync_copy(mnP_hbm.at[0], segB)
    pltpu.sync_copy(mnP_hbm.at[1], tmp)
    _elemwise2(segB, tmp, NP, jnp.minimum)
    _fill(uA, NP, BIG, jnp.int32)
    _fill(uB, NP, BIG, jnp.int32)

    def phaseC(i, _):
        hv = idx_h[pl.ds(i * 16, 16)]
        nv = idx_n[pl.ds(i * 16, 16)]
        sv = plsc.load_gather(s_tab, [nv])
        mx = plsc.load_gather(segA, [hv])
        mn = plsc.load_gather(segB, [hv])
        cand_hi = jnp.where(sv == mx, nv, BIG)
        cand_lo = jnp.where(sv == mn, nv, BIG)
        _winner_rmw(conflict, hv, full, lane,
                    [(uA, cand_hi, jnp.minimum), (uB, cand_lo, jnp.minimum)])
        return 0

    lax.fori_loop(0, EPH2 // 16, phaseC, 0, unroll=4)

    pltpu.sync_copy(uA, shI.at[t])
    plsc.subcore_barrier()
    body = _combine_stripe(shI, sti, t, jnp.minimum, BIG)(strb)
    lax.fori_loop(0, ST // 16, body, 0)
    pltpu.sync_copy(strb, uAP_hbm.at[c, pl.ds(t * ST, ST)])
    plsc.subcore_barrier()
    pltpu.sync_copy(uB, shI.at[t])
    plsc.subcore_barrier()
    body = _combine_stripe(shI, sti, t, jnp.minimum, BIG)(strb)
    lax.fori_loop(0, ST // 16, body, 0)
    pltpu.sync_copy(strb, uBP_hbm.at[c, pl.ds(t * ST, ST)])


def _gbb_body(s_hbm, nidx_hbm, hidx_hbm, mxP_hbm, mnP_hbm,
              uAP_hbm, uBP_hbm, *scratch):
    c = lax.axis_index("c")
    t = lax.axis_index("s")
    _gbb_work(s_hbm, nidx_hbm, hidx_hbm, mxP_hbm, mnP_hbm, uAP_hbm, uBP_hbm,
              *scratch, c, t)


_gb_b = functools.partial(
    pl.kernel,
    out_type=[
        jax.ShapeDtypeStruct((2, NP), jnp.int32),
        jax.ShapeDtypeStruct((2, NP), jnp.int32),
    ],
    mesh=_mesh,
    compiler_params=pltpu.CompilerParams(needs_layout_passes=False),
    scratch_types=[
        pltpu.VMEM((NP,), jnp.float32),
        pltpu.VMEM((EPH2,), jnp.int32),
        pltpu.VMEM((EPH2,), jnp.int32),
        pltpu.VMEM((NP,), jnp.float32),
        pltpu.VMEM((NP,), jnp.float32),
        pltpu.VMEM((NP,), jnp.float32),
        pltpu.VMEM((NP,), jnp.int32),
        pltpu.VMEM((NP,), jnp.int32),
        pltpu.VMEM((NP,), jnp.int32),
        pltpu.VMEM((16, ST), jnp.int32),
        pltpu.VMEM((ST,), jnp.int32),
        pltpu.VMEM_SHARED((16, NP), jnp.int32),
    ],
)(_gbb_body)


def _gbc_work(uAP_hbm, uBP_hbm, ro_hbm, ri_hbm, w_hbm,
              s_tab, segA, conflict, stf, ustrA, ustrB, u2,
              srcb, dstb, diagb, dinvstr, wbuf, shF):
    t = lax.axis_index("s")
    lane = lax.iota(jnp.int32, 16)

    stripe = pl.ds(t * ST, ST)
    pltpu.sync_copy(uAP_hbm.at[0, stripe], ustrA)
    pltpu.sync_copy(uAP_hbm.at[1, stripe], u2)
    _elemwise2(ustrA, u2, ST, jnp.minimum)
    pltpu.sync_copy(uBP_hbm.at[0, stripe], ustrB)
    pltpu.sync_copy(uBP_hbm.at[1, stripe], u2)
    _elemwise2(ustrB, u2, ST, jnp.minimum)

    def phaseD1(j, _):
        ua = ustrA[pl.ds(j * 16, 16)]
        ub = ustrB[pl.ds(j * 16, 16)]
        valid = (ua < BIG) & (ub < BIG) & (ua != ub)
        srcb[pl.ds(j * 16, 16)] = jnp.where(valid, ua, 0)
        dstb[pl.ds(j * 16, 16)] = jnp.where(valid, ub, 0)
        return 0

    lax.fori_loop(0, ST // 16, phaseD1, 0)

    _fill(segA, NP, jnp.float32(0.0), jnp.float32)
    onef = jnp.full((16,), 1.0, jnp.float32)

    def phaseD2(j, _):
        ua = ustrA[pl.ds(j * 16, 16)]
        ub = ustrB[pl.ds(j * 16, 16)]
        sv16 = srcb[pl.ds(j * 16, 16)]
        dv16 = dstb[pl.ds(j * 16, 16)]
        valid = (ua < BIG) & (ub < BIG) & (ua != ub)
        _winner_rmw(conflict, sv16, valid, lane,
                    [(segA, onef, lambda c, v: c + v)])
        _winner_rmw(conflict, dv16, valid, lane,
                    [(segA, onef, lambda c, v: c + v)])
        return 0

    lax.fori_loop(0, ST // 16, phaseD2, 0)

    pltpu.sync_copy(segA, shF.at[t])
    plsc.subcore_barrier()

    def degbody(j, _):
        acc = jnp.full((16,), 0.0, jnp.float32)
        for k in range(16):
            acc = acc + stf[k, pl.ds(j * 16, 16)]
        deg = acc + 1.0
        bits = plsc.bitcast(deg, jnp.int32)
        y = plsc.bitcast(jnp.int32(0x5F3759DF) - (bits >> 1), jnp.float32)
        for _i in range(3):
            y = y * (1.5 - 0.5 * deg * y * y)
        dinvstr[pl.ds(j * 16, 16)] = y
        return 0

    lax.fori_loop(0, ST // 16, degbody, 0)
    pltpu.sync_copy(dinvstr, shF.at[0, pl.ds(t * ST, ST)])
    plsc.subcore_barrier()
    pltpu.sync_copy(shF.at[0], s_tab)
    plsc.subcore_barrier()

    def dsbody(j, _):
        y = dinvstr[pl.ds(j * 16, 16)]
        wbuf[pl.ds(2 * ST + j * 16, 16)] = y * y
        diagb[pl.ds(j * 16, 16)] = t * ST + j * 16 + lane
        return 0

    lax.fori_loop(0, ST // 16, dsbody, 0)

    def wbody(j, _):
        ua = ustrA[pl.ds(j * 16, 16)]
        ub = ustrB[pl.ds(j * 16, 16)]
        sv16 = srcb[pl.ds(j * 16, 16)]
        dv16 = dstb[pl.ds(j * 16, 16)]
        valid = (ua < BIG) & (ub < BIG) & (ua != ub)
        ds_ = plsc.load_gather(s_tab, [sv16])
        dd_ = plsc.load_gather(s_tab, [dv16])
        wv = jnp.where(valid, ds_ * dd_, 0.0)
        wbuf[pl.ds(j * 16, 16)] = wv
        wbuf[pl.ds(ST + j * 16, 16)] = wv
        return 0

    lax.fori_loop(0, ST // 16, wbody, 0)

    pltpu.sync_copy(srcb, ro_hbm.at[pl.ds(t * UPT, ST)])
    pltpu.sync_copy(dstb, ro_hbm.at[pl.ds(t * UPT + ST, ST)])
    pltpu.sync_copy(diagb, ro_hbm.at[pl.ds(t * UPT + 2 * ST, ST)])
    pltpu.sync_copy(dstb, ri_hbm.at[pl.ds(t * UPT, ST)])
    pltpu.sync_copy(srcb, ri_hbm.at[pl.ds(t * UPT + ST, ST)])
    pltpu.sync_copy(diagb, ri_hbm.at[pl.ds(t * UPT + 2 * ST, ST)])
    pltpu.sync_copy(wbuf, w_hbm.at[pl.ds(t * UPT, UPT)])


def _gbc_body(uAP_hbm, uBP_hbm, ro_hbm, ri_hbm, w_hbm, *scratch):
    c = lax.axis_index("c")
    pl.when(c == 0)(lambda: _gbc_work(
        uAP_hbm, uBP_hbm, ro_hbm, ri_hbm, w_hbm, *scratch))


_gb_c = functools.partial(
    pl.kernel,
    out_type=[
        jax.ShapeDtypeStruct((NUPD,), jnp.int32),
        jax.ShapeDtypeStruct((NUPD,), jnp.int32),
        jax.ShapeDtypeStruct((NUPD,), jnp.float32),
    ],
    mesh=_mesh,
    compiler_params=pltpu.CompilerParams(needs_layout_passes=False),
    scratch_types=[
        pltpu.VMEM((NP,), jnp.float32),
        pltpu.VMEM((NP,), jnp.float32),
        pltpu.VMEM((NP,), jnp.int32),
        pltpu.VMEM((16, ST), jnp.float32),
        pltpu.VMEM((ST,), jnp.int32),
        pltpu.VMEM((ST,), jnp.int32),
        pltpu.VMEM((ST,), jnp.int32),
        pltpu.VMEM((ST,), jnp.int32),
        pltpu.VMEM((ST,), jnp.int32),
        pltpu.VMEM((ST,), jnp.int32),
        pltpu.VMEM((ST,), jnp.float32),
        pltpu.VMEM((UPT,), jnp.float32),
        pltpu.VMEM_SHARED((16, NP), jnp.float32),
    ],
)(_gbc_body)


def _spmm_work(h_hbm, o_hbm, ro_hbm, ri_hbm, w_hbm,
               gbuf, rov, riv, wv, acc, semg, sems, t, f2):
    stripe = pl.ds(t * ST, ST)
    pltpu.sync_copy(ro_hbm.at[t], rov)
    pltpu.sync_copy(ri_hbm.at[t], riv)
    pltpu.sync_copy(w_hbm.at[pl.ds(t * UPT, UPT)], wv)

    zero = jnp.full((16,), 0.0, jnp.float32)

    def zbody(i, _):
        gbuf[i // (f2 // 16), pl.ds((i % (f2 // 16)) * 16, 16)] = zero
        return 0

    lax.fori_loop(0, CH * f2 // 16, zbody, 0)
    for blk in range(ST // CH):
        pltpu.sync_copy(gbuf.at[pl.ds(0, CH)],
                        acc.at[pl.ds(t * ST + blk * CH, CH)])
    plsc.subcore_barrier()

    def scale16(jj, _):
        w16 = wv[pl.ds(jj * 16, 16)]
        for k16 in range(16):
            sc = w16[k16]
            row = (jj * 16 + k16) % (NCH_P1 * CH)
            for cc in range(f2 // 16):
                col = pl.ds(cc * 16, 16)
                gbuf[row, col] = gbuf[row, col] * sc
        return 0

    passes = [list(range(0, NCH_P1)), list(range(NCH_P1, NCH))]
    for pi, chunks in enumerate(passes):
        base = chunks[0]
        gathers = [
            pltpu.async_copy(h_hbm.at[riv.at[chunk]],
                             gbuf.at[pl.ds((chunk - base) * CH, CH)], semg)
            for chunk in chunks
        ]
        for g in gathers:
            g.wait()
        lax.fori_loop(base * CH // 16, (chunks[-1] + 1) * CH // 16, scale16, 0)
        scatters = [
            pltpu.async_copy(gbuf.at[pl.ds((chunk - base) * CH, CH)],
                             acc.at[rov.at[chunk]], sems, add=True)
            for chunk in chunks
        ]
        for sctr in scatters:
            sctr.wait()

    plsc.subcore_barrier()
    pltpu.sync_copy(acc.at[stripe], o_hbm.at[stripe])


def _make_spmm(f2):
    def body(ha, hb, ro3, ri3, w, oa, ob,
             gbuf, rov, riv, wv, acc, semg, sems):
        c = lax.axis_index("c")
        t = lax.axis_index("s")
        pl.when(c == 0)(lambda: _spmm_work(
            ha, oa, ro3, ri3, w, gbuf, rov, riv, wv, acc, semg, sems, t, f2))
        pl.when(c == 1)(lambda: _spmm_work(
            hb, ob, ro3, ri3, w, gbuf, rov, riv, wv, acc, semg, sems, t, f2))

    return functools.partial(
        pl.kernel,
        out_type=[
            jax.ShapeDtypeStruct((NP, f2), jnp.float32),
            jax.ShapeDtypeStruct((NP, f2), jnp.float32),
        ],
        mesh=_mesh,
        compiler_params=pltpu.CompilerParams(needs_layout_passes=False,
                                             use_tc_tiling_on_sc=False),
        scratch_types=[
            pltpu.VMEM((NCH_P1 * CH, f2), jnp.float32),
            pltpu.VMEM((NCH, CH), jnp.int32),
            pltpu.VMEM((NCH, CH), jnp.int32),
            pltpu.VMEM((UPT,), jnp.float32),
            pltpu.VMEM_SHARED((NP, f2), jnp.float32),
            pltpu.SemaphoreType.DMA,
            pltpu.SemaphoreType.DMA,
        ],
    )(body)


_spmm64 = _make_spmm(64)
_spmm32 = _make_spmm(32)


def _mm_split_kernel(x_ref, w_ref, b_ref, ha, hb):
    h = jnp.dot(x_ref[...], w_ref[...],
                preferred_element_type=jnp.float32) + b_ref[...]
    half = h.shape[1] // 2
    ha[...] = h[:, :half]
    hb[...] = h[:, half:]


def _mm1(xp, W1, b1):
    blk = 2048
    m = W1.shape[1]
    half = m // 2
    sds = jax.ShapeDtypeStruct((NP, half), jnp.float32)
    return pl.pallas_call(
        _mm_split_kernel,
        grid=(NP // blk,),
        in_specs=[
            pl.BlockSpec((blk, 128), lambda i: (i, 0)),
            pl.BlockSpec((128, m), lambda i: (0, 0)),
            pl.BlockSpec((m,), lambda i: (0,)),
        ],
        out_specs=[pl.BlockSpec((blk, half), lambda i: (i, 0))] * 2,
        out_shape=[sds, sds],
    )(xp, W1, b1)


def _mm2_kernel(a_ref, b_ref, w_ref, bias_ref, oa, ob):
    h = jnp.concatenate([a_ref[...], b_ref[...]], axis=1)
    h = jax.nn.relu(h)
    o = jnp.dot(h, w_ref[...], preferred_element_type=jnp.float32) + bias_ref[...]
    half = o.shape[1] // 2
    oa[...] = o[:, :half]
    ob[...] = o[:, half:]


def _mm2(h1a, h1b, W2, b2):
    blk = 2048
    m = W2.shape[1]
    half = m // 2
    sds = jax.ShapeDtypeStruct((NP, half), jnp.float32)
    return pl.pallas_call(
        _mm2_kernel,
        grid=(NP // blk,),
        in_specs=[
            pl.BlockSpec((blk, 64), lambda i: (i, 0)),
            pl.BlockSpec((blk, 64), lambda i: (i, 0)),
            pl.BlockSpec((128, m), lambda i: (0, 0)),
            pl.BlockSpec((m,), lambda i: (0,)),
        ],
        out_specs=[pl.BlockSpec((blk, half), lambda i: (i, 0))] * 2,
        out_shape=[sds, sds],
    )(h1a, h1b, W2, b2)


def kernel(x, hyperedge_index, r, W1, b1, W2, b2):
    node_idx = hyperedge_index[0]
    he_idx = hyperedge_index[1]
    s = _matvec(x, r)
    s_pad = jnp.pad(s, (0, NP - N_NODES_C))
    mxP, mnP = _gb_a(s_pad, node_idx, he_idx)
    uAP, uBP = _gb_b(s_pad, node_idx, he_idx, mxP, mnP)
    ro, ri, w = _gb_c(uAP, uBP)
    ro3 = ro.reshape(16, NCH, CH)
    ri3 = ri.reshape(16, NCH, CH)

    xp = jnp.pad(x, ((0, NP - N_NODES_C), (0, 0)))
    ha, hb = _mm1(xp, W1, b1)
    o1a, o1b = _spmm64(ha, hb, ro3, ri3, w)
    oa, ob = _mm2(o1a, o1b, W2, b2)
    qa, qb = _spmm32(oa, ob, ro3, ri3, w)
    return jnp.concatenate([qa[:N_NODES_C], qb[:N_NODES_C]], axis=1)


# speedup vs baseline: 30.7024x; 1.0910x over previous
import functools

import jax
import jax.numpy as jnp
from jax import lax
from jax.experimental import pallas as pl
from jax.experimental.pallas import tpu as pltpu
from jax.experimental.pallas import tpu_sc as plsc

N_NODES_C = 10000
N_HE_C = 10000
NNZ_C = 320000
NP = 10240
ST = 640
EPH2 = NNZ_C // 32
UPT = 3 * ST
NUPD = 16 * UPT
CH = 128
NCH = UPT // CH
NCH_P1 = 8
BIG = N_NODES_C
NEGF = -3.0e38
POSF = 3.0e38

_mesh = plsc.VectorSubcoreMesh(core_axis_name="c", subcore_axis_name="s")


def _mm_kernel(x_ref, w_ref, b_ref, o_ref):
    o_ref[...] = jnp.dot(x_ref[...], w_ref[...],
                         preferred_element_type=jnp.float32) + b_ref[...]


def _matmul_bias(x, w, b):
    n, k = x.shape
    m = w.shape[1]
    blk = 2000
    return pl.pallas_call(
        _mm_kernel,
        grid=(n // blk,),
        in_specs=[
            pl.BlockSpec((blk, k), lambda i: (i, 0)),
            pl.BlockSpec((k, m), lambda i: (0, 0)),
            pl.BlockSpec((m,), lambda i: (0,)),
        ],
        out_specs=pl.BlockSpec((blk, m), lambda i: (i, 0)),
        out_shape=jax.ShapeDtypeStruct((n, m), jnp.float32),
    )(x, w, b)


def _matvec_kernel(x_ref, r_ref, o_ref):
    o_ref[...] = jnp.dot(x_ref[...], r_ref[...],
                         preferred_element_type=jnp.float32)


def _matvec(x, r):
    n, k = x.shape
    blk = 2000
    return pl.pallas_call(
        _matvec_kernel,
        grid=(n // blk,),
        in_specs=[
            pl.BlockSpec((blk, k), lambda i: (i, 0)),
            pl.BlockSpec((k, 128), lambda i: (0, 0)),
        ],
        out_specs=pl.BlockSpec((blk, 128), lambda i: (i, 0)),
        out_shape=jax.ShapeDtypeStruct((n, 128), jnp.float32),
    )(x, jnp.tile(r[:, None], (1, 128)))[:, 0]


def _fill(ref, nwords, val, dtype):
    vec = jnp.full((16,), val, dtype)

    def body(i, _):
        ref[pl.ds(i * 16, 16)] = vec
        return 0

    lax.fori_loop(0, nwords // 16, body, 0)


def _winner_rmw(conflict_ref, idx, mask0, lane, updates):
    def cond(pend):
        return jnp.any(pend)

    def body(pend):
        plsc.store_scatter(conflict_ref, [idx], lane, mask=pend)
        win = plsc.load_gather(conflict_ref, [idx], mask=pend)
        wm = pend & (win == lane)
        for ref, val, comb in updates:
            cur = plsc.load_gather(ref, [idx], mask=wm)
            plsc.store_scatter(ref, [idx], comb(cur, val), mask=wm)
        return pend & jnp.logical_not(wm)

    lax.while_loop(cond, body, mask0)


def _combine_stripe(sh, stf, t, op, init, nrefs=16):
    for k in range(nrefs):
        pltpu.sync_copy(sh.at[k, pl.ds(t * ST, ST)], stf.at[k])

    def make_body(out_ref):
        def body(j, _):
            acc = jnp.full((16,), init)
            for k in range(nrefs):
                acc = op(acc, stf[k, pl.ds(j * 16, 16)])
            out_ref[pl.ds(j * 16, 16)] = acc
            return 0
        return body

    return make_body


def _gba_work(s_hbm, nidx_hbm, hidx_hbm, mxP_hbm, mnP_hbm,
              s_tab, idx_n, idx_h, segA, segB, conflict, stf, strb, shF,
              c, t):
    g = c * 16 + t
    lane = lax.iota(jnp.int32, 16)
    full = jnp.full((16,), True)

    pltpu.sync_copy(s_hbm, s_tab)
    pltpu.sync_copy(nidx_hbm.at[pl.ds(g * EPH2, EPH2)], idx_n)
    pltpu.sync_copy(hidx_hbm.at[pl.ds(g * EPH2, EPH2)], idx_h)
    _fill(segA, NP, NEGF, jnp.float32)
    _fill(segB, NP, POSF, jnp.float32)

    def phaseB(i, _):
        hv = idx_h[pl.ds(i * 16, 16)]
        nv = idx_n[pl.ds(i * 16, 16)]
        sv = plsc.load_gather(s_tab, [nv])
        _winner_rmw(conflict, hv, full, lane,
                    [(segA, sv, jnp.maximum), (segB, sv, jnp.minimum)])
        return 0

    lax.fori_loop(0, EPH2 // 16, phaseB, 0, unroll=4)

    pltpu.sync_copy(segA, shF.at[t])
    plsc.subcore_barrier()
    body = _combine_stripe(shF, stf, t, jnp.maximum, NEGF)(strb)
    lax.fori_loop(0, ST // 16, body, 0)
    pltpu.sync_copy(strb, mxP_hbm.at[c, pl.ds(t * ST, ST)])
    plsc.subcore_barrier()
    pltpu.sync_copy(segB, shF.at[t])
    plsc.subcore_barrier()
    body = _combine_stripe(shF, stf, t, jnp.minimum, POSF)(strb)
    lax.fori_loop(0, ST // 16, body, 0)
    pltpu.sync_copy(strb, mnP_hbm.at[c, pl.ds(t * ST, ST)])


def _gba_body(s_hbm, nidx_hbm, hidx_hbm, mxP_hbm, mnP_hbm, *scratch):
    c = lax.axis_index("c")
    t = lax.axis_index("s")
    _gba_work(s_hbm, nidx_hbm, hidx_hbm, mxP_hbm, mnP_hbm, *scratch, c, t)


_gb_a = functools.partial(
    pl.kernel,
    out_type=[
        jax.ShapeDtypeStruct((2, NP), jnp.float32),
        jax.ShapeDtypeStruct((2, NP), jnp.float32),
    ],
    mesh=_mesh,
    compiler_params=pltpu.CompilerParams(needs_layout_passes=False),
    scratch_types=[
        pltpu.VMEM((NP,), jnp.float32),
        pltpu.VMEM((EPH2,), jnp.int32),
        pltpu.VMEM((EPH2,), jnp.int32),
        pltpu.VMEM((NP,), jnp.float32),
        pltpu.VMEM((NP,), jnp.float32),
        pltpu.VMEM((NP,), jnp.int32),
        pltpu.VMEM((16, ST), jnp.float32),
        pltpu.VMEM((ST,), jnp.float32),
        pltpu.VMEM_SHARED((16, NP), jnp.float32),
    ],
)(_gba_body)


def _elemwise2(dst, other, n, op):
    def body(i, _):
        sl = pl.ds(i * 16, 16)
        dst[sl] = op(dst[sl], other[sl])
        return 0

    lax.fori_loop(0, n // 16, body, 0)


def _gbb_work(s_hbm, nidx_hbm, hidx_hbm, mxP_hbm, mnP_hbm, uAP_hbm, uBP_hbm,
              s_tab, idx_n, idx_h, segA, segB, tmp, uA, uB, conflict,
              sti, strb, shI, c, t):
    g = c * 16 + t
    lane = lax.iota(jnp.int32, 16)
    full = jnp.full((16,), True)

    pltpu.sync_copy(s_hbm, s_tab)
    pltpu.sync_copy(nidx_hbm.at[pl.ds(g * EPH2, EPH2)], idx_n)
    pltpu.sync_copy(hidx_hbm.at[pl.ds(g * EPH2, EPH2)], idx_h)
    pltpu.sync_copy(mxP_hbm.at[0], segA)
    pltpu.sync_copy(mxP_hbm.at[1], tmp)
    _elemwise2(segA, tmp, NP, jnp.maximum)
    pltpu.sync_copy(mnP_hbm.at[0], segB)
    pltpu.sync_copy(mnP_hbm.at[1], tmp)
    _elemwise2(segB, tmp, NP, jnp.minimum)
    _fill(uA, NP, BIG, jnp.int32)
    _fill(uB, NP, BIG, jnp.int32)

    def phaseC(i, _):
        hv = idx_h[pl.ds(i * 16, 16)]
        nv = idx_n[pl.ds(i * 16, 16)]
        sv = plsc.load_gather(s_tab, [nv])
        mx = plsc.load_gather(segA, [hv])
        mn = plsc.load_gather(segB, [hv])
        cand_hi = jnp.where(sv == mx, nv, BIG)
        cand_lo = jnp.where(sv == mn, nv, BIG)
        _winner_rmw(conflict, hv, full, lane,
                    [(uA, cand_hi, jnp.minimum), (uB, cand_lo, jnp.minimum)])
        return 0

    lax.fori_loop(0, EPH2 // 16, phaseC, 0, unroll=4)

    pltpu.sync_copy(uA, shI.at[t])
    plsc.subcore_barrier()
    body = _combine_stripe(shI, sti, t, jnp.minimum, BIG)(strb)
    lax.fori_loop(0, ST // 16, body, 0)
    pltpu.sync_copy(strb, uAP_hbm.at[c, pl.ds(t * ST, ST)])
    plsc.subcore_barrier()
    pltpu.sync_copy(uB, shI.at[t])
    plsc.subcore_barrier()
    body = _combine_stripe(shI, sti, t, jnp.minimum, BIG)(strb)
    lax.fori_loop(0, ST // 16, body, 0)
    pltpu.sync_copy(strb, uBP_hbm.at[c, pl.ds(t * ST, ST)])


def _gbb_body(s_hbm, nidx_hbm, hidx_hbm, mxP_hbm, mnP_hbm,
              uAP_hbm, uBP_hbm, *scratch):
    c = lax.axis_index("c")
    t = lax.axis_index("s")
    _gbb_work(s_hbm, nidx_hbm, hidx_hbm, mxP_hbm, mnP_hbm, uAP_hbm, uBP_hbm,
              *scratch, c, t)


_gb_b = functools.partial(
    pl.kernel,
    out_type=[
        jax.ShapeDtypeStruct((2, NP), jnp.int32),
        jax.ShapeDtypeStruct((2, NP), jnp.int32),
    ],
    mesh=_mesh,
    compiler_params=pltpu.CompilerParams(needs_layout_passes=False),
    scratch_types=[
        pltpu.VMEM((NP,), jnp.float32),
        pltpu.VMEM((EPH2,), jnp.int32),
        pltpu.VMEM((EPH2,), jnp.int32),
        pltpu.VMEM((NP,), jnp.float32),
        pltpu.VMEM((NP,), jnp.float32),
        pltpu.VMEM((NP,), jnp.float32),
        pltpu.VMEM((NP,), jnp.int32),
        pltpu.VMEM((NP,), jnp.int32),
        pltpu.VMEM((NP,), jnp.int32),
        pltpu.VMEM((16, ST), jnp.int32),
        pltpu.VMEM((ST,), jnp.int32),
        pltpu.VMEM_SHARED((16, NP), jnp.int32),
    ],
)(_gbb_body)


def _gbc_work(uAP_hbm, uBP_hbm, ro_hbm, ri_hbm, w_hbm,
              s_tab, segA, conflict, stf, ustrA, ustrB, u2,
              srcb, dstb, diagb, dinvstr, wbuf, shF):
    t = lax.axis_index("s")
    lane = lax.iota(jnp.int32, 16)

    stripe = pl.ds(t * ST, ST)
    pltpu.sync_copy(uAP_hbm.at[0, stripe], ustrA)
    pltpu.sync_copy(uAP_hbm.at[1, stripe], u2)
    _elemwise2(ustrA, u2, ST, jnp.minimum)
    pltpu.sync_copy(uBP_hbm.at[0, stripe], ustrB)
    pltpu.sync_copy(uBP_hbm.at[1, stripe], u2)
    _elemwise2(ustrB, u2, ST, jnp.minimum)

    def phaseD1(j, _):
        ua = ustrA[pl.ds(j * 16, 16)]
        ub = ustrB[pl.ds(j * 16, 16)]
        valid = (ua < BIG) & (ub < BIG) & (ua != ub)
        srcb[pl.ds(j * 16, 16)] = jnp.where(valid, ua, 0)
        dstb[pl.ds(j * 16, 16)] = jnp.where(valid, ub, 0)
        return 0

    lax.fori_loop(0, ST // 16, phaseD1, 0)

    _fill(segA, NP, jnp.float32(0.0), jnp.float32)
    onef = jnp.full((16,), 1.0, jnp.float32)

    def phaseD2(j, _):
        ua = ustrA[pl.ds(j * 16, 16)]
        ub = ustrB[pl.ds(j * 16, 16)]
        sv16 = srcb[pl.ds(j * 16, 16)]
        dv16 = dstb[pl.ds(j * 16, 16)]
        valid = (ua < BIG) & (ub < BIG) & (ua != ub)
        _winner_rmw(conflict, sv16, valid, lane,
                    [(segA, onef, lambda c, v: c + v)])
        _winner_rmw(conflict, dv16, valid, lane,
                    [(segA, onef, lambda c, v: c + v)])
        return 0

    lax.fori_loop(0, ST // 16, phaseD2, 0)

    pltpu.sync_copy(segA, shF.at[t])
    plsc.subcore_barrier()

    def degbody(j, _):
        acc = jnp.full((16,), 0.0, jnp.float32)
        for k in range(16):
            acc = acc + stf[k, pl.ds(j * 16, 16)]
        deg = acc + 1.0
        bits = plsc.bitcast(deg, jnp.int32)
        y = plsc.bitcast(jnp.int32(0x5F3759DF) - (bits >> 1), jnp.float32)
        for _i in range(3):
            y = y * (1.5 - 0.5 * deg * y * y)
        dinvstr[pl.ds(j * 16, 16)] = y
        return 0

    lax.fori_loop(0, ST // 16, degbody, 0)
    pltpu.sync_copy(dinvstr, shF.at[0, pl.ds(t * ST, ST)])
    plsc.subcore_barrier()
    pltpu.sync_copy(shF.at[0], s_tab)
    plsc.subcore_barrier()

    def dsbody(j, _):
        y = dinvstr[pl.ds(j * 16, 16)]
        wbuf[pl.ds(2 * ST + j * 16, 16)] = y * y
        diagb[pl.ds(j * 16, 16)] = t * ST + j * 16 + lane
        return 0

    lax.fori_loop(0, ST // 16, dsbody, 0)

    def wbody(j, _):
        ua = ustrA[pl.ds(j * 16, 16)]
        ub = ustrB[pl.ds(j * 16, 16)]
        sv16 = srcb[pl.ds(j * 16, 16)]
        dv16 = dstb[pl.ds(j * 16, 16)]
        valid = (ua < BIG) & (ub < BIG) & (ua != ub)
        ds_ = plsc.load_gather(s_tab, [sv16])
        dd_ = plsc.load_gather(s_tab, [dv16])
        wv = jnp.where(valid, ds_ * dd_, 0.0)
        wbuf[pl.ds(j * 16, 16)] = wv
        wbuf[pl.ds(ST + j * 16, 16)] = wv
        return 0

    lax.fori_loop(0, ST // 16, wbody, 0)

    pltpu.sync_copy(srcb, ro_hbm.at[pl.ds(t * UPT, ST)])
    pltpu.sync_copy(dstb, ro_hbm.at[pl.ds(t * UPT + ST, ST)])
    pltpu.sync_copy(diagb, ro_hbm.at[pl.ds(t * UPT + 2 * ST, ST)])
    pltpu.sync_copy(dstb, ri_hbm.at[pl.ds(t * UPT, ST)])
    pltpu.sync_copy(srcb, ri_hbm.at[pl.ds(t * UPT + ST, ST)])
    pltpu.sync_copy(diagb, ri_hbm.at[pl.ds(t * UPT + 2 * ST, ST)])
    pltpu.sync_copy(wbuf, w_hbm.at[pl.ds(t * UPT, UPT)])


def _gbc_body(uAP_hbm, uBP_hbm, ro_hbm, ri_hbm, w_hbm, *scratch):
    c = lax.axis_index("c")
    pl.when(c == 0)(lambda: _gbc_work(
        uAP_hbm, uBP_hbm, ro_hbm, ri_hbm, w_hbm, *scratch))


_gb_c = functools.partial(
    pl.kernel,
    out_type=[
        jax.ShapeDtypeStruct((NUPD,), jnp.int32),
        jax.ShapeDtypeStruct((NUPD,), jnp.int32),
        jax.ShapeDtypeStruct((NUPD,), jnp.float32),
    ],
    mesh=_mesh,
    compiler_params=pltpu.CompilerParams(needs_layout_passes=False),
    scratch_types=[
        pltpu.VMEM((NP,), jnp.float32),
        pltpu.VMEM((NP,), jnp.float32),
        pltpu.VMEM((NP,), jnp.int32),
        pltpu.VMEM((16, ST), jnp.float32),
        pltpu.VMEM((ST,), jnp.int32),
        pltpu.VMEM((ST,), jnp.int32),
        pltpu.VMEM((ST,), jnp.int32),
        pltpu.VMEM((ST,), jnp.int32),
        pltpu.VMEM((ST,), jnp.int32),
        pltpu.VMEM((ST,), jnp.int32),
        pltpu.VMEM((ST,), jnp.float32),
        pltpu.VMEM((UPT,), jnp.float32),
        pltpu.VMEM_SHARED((16, NP), jnp.float32),
    ],
)(_gbc_body)


def _spmm_work(h_hbm, o_hbm, ro_hbm, ri_hbm, w_hbm,
               gbuf, rov, riv, wv, acc, semg, sems, t, f2):
    stripe = pl.ds(t * ST, ST)
    pltpu.sync_copy(ro_hbm.at[t], rov)
    pltpu.sync_copy(ri_hbm.at[t], riv)
    pltpu.sync_copy(w_hbm.at[pl.ds(t * UPT, UPT)], wv)

    def make_scale(base):
        def scale16(jj, _):
            w16 = wv[pl.ds(jj * 16, 16)]
            for k16 in range(16):
                sc = w16[k16]
                row = jj * 16 + k16 - base * CH
                for cc in range(f2 // 16):
                    col = pl.ds(cc * 16, 16)
                    gbuf[row, col] = gbuf[row, col] * sc
            return 0
        return scale16

    def run_pass(chunks, add):
        base = chunks[0]
        gathers = [
            pltpu.async_copy(h_hbm.at[riv.at[chunk]],
                             gbuf.at[pl.ds((chunk - base) * CH, CH)], semg)
            for chunk in chunks
        ]
        for g in gathers:
            g.wait()
        lax.fori_loop(base * CH // 16, (chunks[-1] + 1) * CH // 16,
                      make_scale(base), 0)
        scatters = [
            pltpu.async_copy(gbuf.at[pl.ds((chunk - base) * CH, CH)],
                             acc.at[rov.at[chunk]], sems, add=add)
            for chunk in chunks
        ]
        for sctr in scatters:
            sctr.wait()

    run_pass(list(range(2 * ST // CH, NCH)), add=False)
    plsc.subcore_barrier()
    run_pass(list(range(0, NCH_P1)), add=True)
    run_pass(list(range(NCH_P1, 2 * ST // CH)), add=True)

    plsc.subcore_barrier()
    pltpu.sync_copy(acc.at[stripe], o_hbm.at[stripe])


def _make_spmm(f2):
    def body(ha, hb, ro3, ri3, w, oa, ob,
             gbuf, rov, riv, wv, acc, semg, sems):
        c = lax.axis_index("c")
        t = lax.axis_index("s")
        pl.when(c == 0)(lambda: _spmm_work(
            ha, oa, ro3, ri3, w, gbuf, rov, riv, wv, acc, semg, sems, t, f2))
        pl.when(c == 1)(lambda: _spmm_work(
            hb, ob, ro3, ri3, w, gbuf, rov, riv, wv, acc, semg, sems, t, f2))

    return functools.partial(
        pl.kernel,
        out_type=[
            jax.ShapeDtypeStruct((NP, f2), jnp.float32),
            jax.ShapeDtypeStruct((NP, f2), jnp.float32),
        ],
        mesh=_mesh,
        compiler_params=pltpu.CompilerParams(needs_layout_passes=False,
                                             use_tc_tiling_on_sc=False),
        scratch_types=[
            pltpu.VMEM((NCH_P1 * CH, f2), jnp.float32),
            pltpu.VMEM((NCH, CH), jnp.int32),
            pltpu.VMEM((NCH, CH), jnp.int32),
            pltpu.VMEM((UPT,), jnp.float32),
            pltpu.VMEM_SHARED((NP, f2), jnp.float32),
            pltpu.SemaphoreType.DMA,
            pltpu.SemaphoreType.DMA,
        ],
    )(body)


_spmm64 = _make_spmm(64)
_spmm32 = _make_spmm(32)


def _mm_split_kernel(x_ref, w_ref, b_ref, ha, hb):
    h = jnp.dot(x_ref[...], w_ref[...],
                preferred_element_type=jnp.float32) + b_ref[...]
    half = h.shape[1] // 2
    ha[...] = h[:, :half]
    hb[...] = h[:, half:]


def _mm1(xp, W1, b1):
    blk = 2048
    m = W1.shape[1]
    half = m // 2
    sds = jax.ShapeDtypeStruct((NP, half), jnp.float32)
    return pl.pallas_call(
        _mm_split_kernel,
        grid=(NP // blk,),
        in_specs=[
            pl.BlockSpec((blk, 128), lambda i: (i, 0)),
            pl.BlockSpec((128, m), lambda i: (0, 0)),
            pl.BlockSpec((m,), lambda i: (0,)),
        ],
        out_specs=[pl.BlockSpec((blk, half), lambda i: (i, 0))] * 2,
        out_shape=[sds, sds],
    )(xp, W1, b1)


def _mm2_kernel(a_ref, b_ref, w_ref, bias_ref, oa, ob):
    h = jnp.concatenate([a_ref[...], b_ref[...]], axis=1)
    h = jax.nn.relu(h)
    o = jnp.dot(h, w_ref[...], preferred_element_type=jnp.float32) + bias_ref[...]
    half = o.shape[1] // 2
    oa[...] = o[:, :half]
    ob[...] = o[:, half:]


def _mm2(h1a, h1b, W2, b2):
    blk = 2048
    m = W2.shape[1]
    half = m // 2
    sds = jax.ShapeDtypeStruct((NP, half), jnp.float32)
    return pl.pallas_call(
        _mm2_kernel,
        grid=(NP // blk,),
        in_specs=[
            pl.BlockSpec((blk, 64), lambda i: (i, 0)),
            pl.BlockSpec((blk, 64), lambda i: (i, 0)),
            pl.BlockSpec((128, m), lambda i: (0, 0)),
            pl.BlockSpec((m,), lambda i: (0,)),
        ],
        out_specs=[pl.BlockSpec((blk, half), lambda i: (i, 0))] * 2,
        out_shape=[sds, sds],
    )(h1a, h1b, W2, b2)


def kernel(x, hyperedge_index, r, W1, b1, W2, b2):
    node_idx = hyperedge_index[0]
    he_idx = hyperedge_index[1]
    s = _matvec(x, r)
    s_pad = jnp.pad(s, (0, NP - N_NODES_C))
    mxP, mnP = _gb_a(s_pad, node_idx, he_idx)
    uAP, uBP = _gb_b(s_pad, node_idx, he_idx, mxP, mnP)
    ro, ri, w = _gb_c(uAP, uBP)
    ro3 = ro.reshape(16, NCH, CH)
    ri3 = ri.reshape(16, NCH, CH)

    xp = jnp.pad(x, ((0, NP - N_NODES_C), (0, 0)))
    ha, hb = _mm1(xp, W1, b1)
    o1a, o1b = _spmm64(ha, hb, ro3, ri3, w)
    oa, ob = _mm2(o1a, o1b, W2, b2)
    qa, qb = _spmm32(oa, ob, ro3, ri3, w)
    return jnp.concatenate([qa[:N_NODES_C], qb[:N_NODES_C]], axis=1)


# speedup vs baseline: 31.3914x; 1.0224x over previous
import functools

import jax
import jax.numpy as jnp
from jax import lax
from jax.experimental import pallas as pl
from jax.experimental.pallas import tpu as pltpu
from jax.experimental.pallas import tpu_sc as plsc

N_NODES_C = 10000
N_HE_C = 10000
NNZ_C = 320000
NP = 10240
ST = 640
EPH2 = NNZ_C // 32
UPT = 3 * ST
NUPD = 16 * UPT
CH = 128
NCH = UPT // CH
NCH_P1 = 8
BIG = N_NODES_C
NEGF = -3.0e38
POSF = 3.0e38

_mesh = plsc.VectorSubcoreMesh(core_axis_name="c", subcore_axis_name="s")


def _mm_kernel(x_ref, w_ref, b_ref, o_ref):
    o_ref[...] = jnp.dot(x_ref[...], w_ref[...],
                         preferred_element_type=jnp.float32) + b_ref[...]


def _matmul_bias(x, w, b):
    n, k = x.shape
    m = w.shape[1]
    blk = 2000
    return pl.pallas_call(
        _mm_kernel,
        grid=(n // blk,),
        in_specs=[
            pl.BlockSpec((blk, k), lambda i: (i, 0)),
            pl.BlockSpec((k, m), lambda i: (0, 0)),
            pl.BlockSpec((m,), lambda i: (0,)),
        ],
        out_specs=pl.BlockSpec((blk, m), lambda i: (i, 0)),
        out_shape=jax.ShapeDtypeStruct((n, m), jnp.float32),
    )(x, w, b)


def _matvec_kernel(x_ref, r_ref, o_ref):
    o_ref[...] = jnp.dot(x_ref[...], r_ref[...],
                         preferred_element_type=jnp.float32)


def _matvec(x, r):
    n, k = x.shape
    blk = 2000
    return pl.pallas_call(
        _matvec_kernel,
        grid=(n // blk,),
        in_specs=[
            pl.BlockSpec((blk, k), lambda i: (i, 0)),
            pl.BlockSpec((k, 128), lambda i: (0, 0)),
        ],
        out_specs=pl.BlockSpec((blk, 128), lambda i: (i, 0)),
        out_shape=jax.ShapeDtypeStruct((n, 128), jnp.float32),
    )(x, jnp.tile(r[:, None], (1, 128)))[:, 0]


def _fill(ref, nwords, val, dtype):
    vec = jnp.full((16,), val, dtype)

    def body(i, _):
        ref[pl.ds(i * 16, 16)] = vec
        return 0

    lax.fori_loop(0, nwords // 16, body, 0)


def _winner_rmw(conflict_ref, idx, mask0, lane, updates):
    def cond(pend):
        return jnp.any(pend)

    def body(pend):
        plsc.store_scatter(conflict_ref, [idx], lane, mask=pend)
        win = plsc.load_gather(conflict_ref, [idx], mask=pend)
        wm = pend & (win == lane)
        for ref, val, comb in updates:
            cur = plsc.load_gather(ref, [idx], mask=wm)
            plsc.store_scatter(ref, [idx], comb(cur, val), mask=wm)
        return pend & jnp.logical_not(wm)

    lax.while_loop(cond, body, mask0)


def _combine_stripe(sh, stf, t, op, init, nrefs=16, sem=None):
    if sem is not None:
        copies = [
            pltpu.async_copy(sh.at[k, pl.ds(t * ST, ST)], stf.at[k], sem)
            for k in range(nrefs)
        ]
        for cp in copies:
            cp.wait()
    else:
        for k in range(nrefs):
            pltpu.sync_copy(sh.at[k, pl.ds(t * ST, ST)], stf.at[k])

    def make_body(out_ref):
        def body(j, _):
            acc = jnp.full((16,), init)
            for k in range(nrefs):
                acc = op(acc, stf[k, pl.ds(j * 16, 16)])
            out_ref[pl.ds(j * 16, 16)] = acc
            return 0
        return body

    return make_body


def _gba_work(s_hbm, nidx_hbm, hidx_hbm, mxP_hbm, mnP_hbm,
              s_tab, idx_n, idx_h, segA, segB, conflict, stf, strb, shF,
              sem, c, t):
    g = c * 16 + t
    lane = lax.iota(jnp.int32, 16)
    full = jnp.full((16,), True)

    pltpu.sync_copy(s_hbm, s_tab)
    pltpu.sync_copy(nidx_hbm.at[pl.ds(g * EPH2, EPH2)], idx_n)
    pltpu.sync_copy(hidx_hbm.at[pl.ds(g * EPH2, EPH2)], idx_h)
    _fill(segA, NP, NEGF, jnp.float32)
    _fill(segB, NP, POSF, jnp.float32)

    def phaseB(i, _):
        hv = idx_h[pl.ds(i * 16, 16)]
        nv = idx_n[pl.ds(i * 16, 16)]
        sv = plsc.load_gather(s_tab, [nv])
        _winner_rmw(conflict, hv, full, lane,
                    [(segA, sv, jnp.maximum), (segB, sv, jnp.minimum)])
        return 0

    lax.fori_loop(0, EPH2 // 16, phaseB, 0, unroll=4)

    pltpu.sync_copy(segA, shF.at[t])
    plsc.subcore_barrier()
    body = _combine_stripe(shF, stf, t, jnp.maximum, NEGF, sem=sem)(strb)
    lax.fori_loop(0, ST // 16, body, 0)
    pltpu.sync_copy(strb, mxP_hbm.at[c, pl.ds(t * ST, ST)])
    plsc.subcore_barrier()
    pltpu.sync_copy(segB, shF.at[t])
    plsc.subcore_barrier()
    body = _combine_stripe(shF, stf, t, jnp.minimum, POSF, sem=sem)(strb)
    lax.fori_loop(0, ST // 16, body, 0)
    pltpu.sync_copy(strb, mnP_hbm.at[c, pl.ds(t * ST, ST)])


def _gba_body(s_hbm, nidx_hbm, hidx_hbm, mxP_hbm, mnP_hbm, *scratch):
    c = lax.axis_index("c")
    t = lax.axis_index("s")
    _gba_work(s_hbm, nidx_hbm, hidx_hbm, mxP_hbm, mnP_hbm, *scratch, c, t)


_gb_a = functools.partial(
    pl.kernel,
    out_type=[
        jax.ShapeDtypeStruct((2, NP), jnp.float32),
        jax.ShapeDtypeStruct((2, NP), jnp.float32),
    ],
    mesh=_mesh,
    compiler_params=pltpu.CompilerParams(needs_layout_passes=False),
    scratch_types=[
        pltpu.VMEM((NP,), jnp.float32),
        pltpu.VMEM((EPH2,), jnp.int32),
        pltpu.VMEM((EPH2,), jnp.int32),
        pltpu.VMEM((NP,), jnp.float32),
        pltpu.VMEM((NP,), jnp.float32),
        pltpu.VMEM((NP,), jnp.int32),
        pltpu.VMEM((16, ST), jnp.float32),
        pltpu.VMEM((ST,), jnp.float32),
        pltpu.VMEM_SHARED((16, NP), jnp.float32),
        pltpu.SemaphoreType.DMA,
    ],
)(_gba_body)


def _elemwise2(dst, other, n, op):
    def body(i, _):
        sl = pl.ds(i * 16, 16)
        dst[sl] = op(dst[sl], other[sl])
        return 0

    lax.fori_loop(0, n // 16, body, 0)


def _gbb_work(s_hbm, nidx_hbm, hidx_hbm, mxP_hbm, mnP_hbm, uAP_hbm, uBP_hbm,
              s_tab, idx_n, idx_h, segA, segB, tmp, uA, uB, conflict,
              sti, strb, shI, sem, c, t):
    g = c * 16 + t
    lane = lax.iota(jnp.int32, 16)
    full = jnp.full((16,), True)

    pltpu.sync_copy(s_hbm, s_tab)
    pltpu.sync_copy(nidx_hbm.at[pl.ds(g * EPH2, EPH2)], idx_n)
    pltpu.sync_copy(hidx_hbm.at[pl.ds(g * EPH2, EPH2)], idx_h)
    pltpu.sync_copy(mxP_hbm.at[0], segA)
    pltpu.sync_copy(mxP_hbm.at[1], tmp)
    _elemwise2(segA, tmp, NP, jnp.maximum)
    pltpu.sync_copy(mnP_hbm.at[0], segB)
    pltpu.sync_copy(mnP_hbm.at[1], tmp)
    _elemwise2(segB, tmp, NP, jnp.minimum)
    _fill(uA, NP, BIG, jnp.int32)
    _fill(uB, NP, BIG, jnp.int32)

    def phaseC(i, _):
        hv = idx_h[pl.ds(i * 16, 16)]
        nv = idx_n[pl.ds(i * 16, 16)]
        sv = plsc.load_gather(s_tab, [nv])
        mx = plsc.load_gather(segA, [hv])
        mn = plsc.load_gather(segB, [hv])
        cand_hi = jnp.where(sv == mx, nv, BIG)
        cand_lo = jnp.where(sv == mn, nv, BIG)
        _winner_rmw(conflict, hv, full, lane,
                    [(uA, cand_hi, jnp.minimum), (uB, cand_lo, jnp.minimum)])
        return 0

    lax.fori_loop(0, EPH2 // 16, phaseC, 0, unroll=4)

    pltpu.sync_copy(uA, shI.at[t])
    plsc.subcore_barrier()
    body = _combine_stripe(shI, sti, t, jnp.minimum, BIG, sem=sem)(strb)
    lax.fori_loop(0, ST // 16, body, 0)
    pltpu.sync_copy(strb, uAP_hbm.at[c, pl.ds(t * ST, ST)])
    plsc.subcore_barrier()
    pltpu.sync_copy(uB, shI.at[t])
    plsc.subcore_barrier()
    body = _combine_stripe(shI, sti, t, jnp.minimum, BIG, sem=sem)(strb)
    lax.fori_loop(0, ST // 16, body, 0)
    pltpu.sync_copy(strb, uBP_hbm.at[c, pl.ds(t * ST, ST)])


def _gbb_body(s_hbm, nidx_hbm, hidx_hbm, mxP_hbm, mnP_hbm,
              uAP_hbm, uBP_hbm, *scratch):
    c = lax.axis_index("c")
    t = lax.axis_index("s")
    _gbb_work(s_hbm, nidx_hbm, hidx_hbm, mxP_hbm, mnP_hbm, uAP_hbm, uBP_hbm,
              *scratch, c, t)


_gb_b = functools.partial(
    pl.kernel,
    out_type=[
        jax.ShapeDtypeStruct((2, NP), jnp.int32),
        jax.ShapeDtypeStruct((2, NP), jnp.int32),
    ],
    mesh=_mesh,
    compiler_params=pltpu.CompilerParams(needs_layout_passes=False),
    scratch_types=[
        pltpu.VMEM((NP,), jnp.float32),
        pltpu.VMEM((EPH2,), jnp.int32),
        pltpu.VMEM((EPH2,), jnp.int32),
        pltpu.VMEM((NP,), jnp.float32),
        pltpu.VMEM((NP,), jnp.float32),
        pltpu.VMEM((NP,), jnp.float32),
        pltpu.VMEM((NP,), jnp.int32),
        pltpu.VMEM((NP,), jnp.int32),
        pltpu.VMEM((NP,), jnp.int32),
        pltpu.VMEM((16, ST), jnp.int32),
        pltpu.VMEM((ST,), jnp.int32),
        pltpu.VMEM_SHARED((16, NP), jnp.int32),
        pltpu.SemaphoreType.DMA,
    ],
)(_gbb_body)


def _gbc_work(uAP_hbm, uBP_hbm, ro_hbm, ri_hbm, w_hbm,
              s_tab, segA, conflict, stf, ustrA, ustrB, u2,
              srcb, dstb, diagb, dinvstr, wbuf, shF):
    t = lax.axis_index("s")
    lane = lax.iota(jnp.int32, 16)

    stripe = pl.ds(t * ST, ST)
    pltpu.sync_copy(uAP_hbm.at[0, stripe], ustrA)
    pltpu.sync_copy(uAP_hbm.at[1, stripe], u2)
    _elemwise2(ustrA, u2, ST, jnp.minimum)
    pltpu.sync_copy(uBP_hbm.at[0, stripe], ustrB)
    pltpu.sync_copy(uBP_hbm.at[1, stripe], u2)
    _elemwise2(ustrB, u2, ST, jnp.minimum)

    def phaseD1(j, _):
        ua = ustrA[pl.ds(j * 16, 16)]
        ub = ustrB[pl.ds(j * 16, 16)]
        valid = (ua < BIG) & (ub < BIG) & (ua != ub)
        srcb[pl.ds(j * 16, 16)] = jnp.where(valid, ua, 0)
        dstb[pl.ds(j * 16, 16)] = jnp.where(valid, ub, 0)
        return 0

    lax.fori_loop(0, ST // 16, phaseD1, 0)

    _fill(segA, NP, jnp.float32(0.0), jnp.float32)
    onef = jnp.full((16,), 1.0, jnp.float32)

    def phaseD2(j, _):
        ua = ustrA[pl.ds(j * 16, 16)]
        ub = ustrB[pl.ds(j * 16, 16)]
        sv16 = srcb[pl.ds(j * 16, 16)]
        dv16 = dstb[pl.ds(j * 16, 16)]
        valid = (ua < BIG) & (ub < BIG) & (ua != ub)
        _winner_rmw(conflict, sv16, valid, lane,
                    [(segA, onef, lambda c, v: c + v)])
        _winner_rmw(conflict, dv16, valid, lane,
                    [(segA, onef, lambda c, v: c + v)])
        return 0

    lax.fori_loop(0, ST // 16, phaseD2, 0)

    pltpu.sync_copy(segA, shF.at[t])
    plsc.subcore_barrier()

    def degbody(j, _):
        acc = jnp.full((16,), 0.0, jnp.float32)
        for k in range(16):
            acc = acc + stf[k, pl.ds(j * 16, 16)]
        deg = acc + 1.0
        bits = plsc.bitcast(deg, jnp.int32)
        y = plsc.bitcast(jnp.int32(0x5F3759DF) - (bits >> 1), jnp.float32)
        for _i in range(3):
            y = y * (1.5 - 0.5 * deg * y * y)
        dinvstr[pl.ds(j * 16, 16)] = y
        return 0

    lax.fori_loop(0, ST // 16, degbody, 0)
    pltpu.sync_copy(dinvstr, shF.at[0, pl.ds(t * ST, ST)])
    plsc.subcore_barrier()
    pltpu.sync_copy(shF.at[0], s_tab)
    plsc.subcore_barrier()

    def dsbody(j, _):
        y = dinvstr[pl.ds(j * 16, 16)]
        wbuf[pl.ds(2 * ST + j * 16, 16)] = y * y
        diagb[pl.ds(j * 16, 16)] = t * ST + j * 16 + lane
        return 0

    lax.fori_loop(0, ST // 16, dsbody, 0)

    def wbody(j, _):
        ua = ustrA[pl.ds(j * 16, 16)]
        ub = ustrB[pl.ds(j * 16, 16)]
        sv16 = srcb[pl.ds(j * 16, 16)]
        dv16 = dstb[pl.ds(j * 16, 16)]
        valid = (ua < BIG) & (ub < BIG) & (ua != ub)
        ds_ = plsc.load_gather(s_tab, [sv16])
        dd_ = plsc.load_gather(s_tab, [dv16])
        wv = jnp.where(valid, ds_ * dd_, 0.0)
        wbuf[pl.ds(j * 16, 16)] = wv
        wbuf[pl.ds(ST + j * 16, 16)] = wv
        return 0

    lax.fori_loop(0, ST // 16, wbody, 0)

    pltpu.sync_copy(srcb, ro_hbm.at[pl.ds(t * UPT, ST)])
    pltpu.sync_copy(dstb, ro_hbm.at[pl.ds(t * UPT + ST, ST)])
    pltpu.sync_copy(diagb, ro_hbm.at[pl.ds(t * UPT + 2 * ST, ST)])
    pltpu.sync_copy(dstb, ri_hbm.at[pl.ds(t * UPT, ST)])
    pltpu.sync_copy(srcb, ri_hbm.at[pl.ds(t * UPT + ST, ST)])
    pltpu.sync_copy(diagb, ri_hbm.at[pl.ds(t * UPT + 2 * ST, ST)])
    pltpu.sync_copy(wbuf, w_hbm.at[pl.ds(t * UPT, UPT)])


def _gbc_body(uAP_hbm, uBP_hbm, ro_hbm, ri_hbm, w_hbm, *scratch):
    c = lax.axis_index("c")
    pl.when(c == 0)(lambda: _gbc_work(
        uAP_hbm, uBP_hbm, ro_hbm, ri_hbm, w_hbm, *scratch))


_gb_c = functools.partial(
    pl.kernel,
    out_type=[
        jax.ShapeDtypeStruct((NUPD,), jnp.int32),
        jax.ShapeDtypeStruct((NUPD,), jnp.int32),
        jax.ShapeDtypeStruct((NUPD,), jnp.float32),
    ],
    mesh=_mesh,
    compiler_params=pltpu.CompilerParams(needs_layout_passes=False),
    scratch_types=[
        pltpu.VMEM((NP,), jnp.float32),
        pltpu.VMEM((NP,), jnp.float32),
        pltpu.VMEM((NP,), jnp.int32),
        pltpu.VMEM((16, ST), jnp.float32),
        pltpu.VMEM((ST,), jnp.int32),
        pltpu.VMEM((ST,), jnp.int32),
        pltpu.VMEM((ST,), jnp.int32),
        pltpu.VMEM((ST,), jnp.int32),
        pltpu.VMEM((ST,), jnp.int32),
        pltpu.VMEM((ST,), jnp.int32),
        pltpu.VMEM((ST,), jnp.float32),
        pltpu.VMEM((UPT,), jnp.float32),
        pltpu.VMEM_SHARED((16, NP), jnp.float32),
    ],
)(_gbc_body)


def _spmm_work(h_hbm, o_hbm, ro_hbm, ri_hbm, w_hbm,
               gbuf, rov, riv, wv, acc, semg, sems, t, f2):
    stripe = pl.ds(t * ST, ST)
    pltpu.sync_copy(ro_hbm.at[t], rov)
    pltpu.sync_copy(ri_hbm.at[t], riv)
    pltpu.sync_copy(w_hbm.at[pl.ds(t * UPT, UPT)], wv)

    def make_scale(base):
        def scale16(jj, _):
            w16 = wv[pl.ds(jj * 16, 16)]
            for k16 in range(16):
                sc = w16[k16]
                row = jj * 16 + k16 - base * CH
                for cc in range(f2 // 16):
                    col = pl.ds(cc * 16, 16)
                    gbuf[row, col] = gbuf[row, col] * sc
            return 0
        return scale16

    def run_pass(chunks, add):
        base = chunks[0]
        gathers = [
            pltpu.async_copy(h_hbm.at[riv.at[chunk]],
                             gbuf.at[pl.ds((chunk - base) * CH, CH)], semg)
            for chunk in chunks
        ]
        for g in gathers:
            g.wait()
        lax.fori_loop(base * CH // 16, (chunks[-1] + 1) * CH // 16,
                      make_scale(base), 0)
        scatters = [
            pltpu.async_copy(gbuf.at[pl.ds((chunk - base) * CH, CH)],
                             acc.at[rov.at[chunk]], sems, add=add)
            for chunk in chunks
        ]
        for sctr in scatters:
            sctr.wait()

    run_pass(list(range(2 * ST // CH, NCH)), add=False)
    plsc.subcore_barrier()
    run_pass(list(range(0, NCH_P1)), add=True)
    run_pass(list(range(NCH_P1, 2 * ST // CH)), add=True)

    plsc.subcore_barrier()
    pltpu.sync_copy(acc.at[stripe], o_hbm.at[stripe])


def _make_spmm(f2):
    def body(ha, hb, ro3, ri3, w, oa, ob,
             gbuf, rov, riv, wv, acc, semg, sems):
        c = lax.axis_index("c")
        t = lax.axis_index("s")
        pl.when(c == 0)(lambda: _spmm_work(
            ha, oa, ro3, ri3, w, gbuf, rov, riv, wv, acc, semg, sems, t, f2))
        pl.when(c == 1)(lambda: _spmm_work(
            hb, ob, ro3, ri3, w, gbuf, rov, riv, wv, acc, semg, sems, t, f2))

    return functools.partial(
        pl.kernel,
        out_type=[
            jax.ShapeDtypeStruct((NP, f2), jnp.float32),
            jax.ShapeDtypeStruct((NP, f2), jnp.float32),
        ],
        mesh=_mesh,
        compiler_params=pltpu.CompilerParams(needs_layout_passes=False,
                                             use_tc_tiling_on_sc=False),
        scratch_types=[
            pltpu.VMEM((NCH_P1 * CH, f2), jnp.float32),
            pltpu.VMEM((NCH, CH), jnp.int32),
            pltpu.VMEM((NCH, CH), jnp.int32),
            pltpu.VMEM((UPT,), jnp.float32),
            pltpu.VMEM_SHARED((NP, f2), jnp.float32),
            pltpu.SemaphoreType.DMA,
            pltpu.SemaphoreType.DMA,
        ],
    )(body)


_spmm64 = _make_spmm(64)
_spmm32 = _make_spmm(32)


def _mm_split_kernel(x_ref, w_ref, b_ref, ha, hb):
    h = jnp.dot(x_ref[...], w_ref[...],
                preferred_element_type=jnp.float32) + b_ref[...]
    half = h.shape[1] // 2
    ha[...] = h[:, :half]
    hb[...] = h[:, half:]


def _mm1(xp, W1, b1):
    blk = 2048
    m = W1.shape[1]
    half = m // 2
    sds = jax.ShapeDtypeStruct((NP, half), jnp.float32)
    return pl.pallas_call(
        _mm_split_kernel,
        grid=(NP // blk,),
        in_specs=[
            pl.BlockSpec((blk, 128), lambda i: (i, 0)),
            pl.BlockSpec((128, m), lambda i: (0, 0)),
            pl.BlockSpec((m,), lambda i: (0,)),
        ],
        out_specs=[pl.BlockSpec((blk, half), lambda i: (i, 0))] * 2,
        out_shape=[sds, sds],
    )(xp, W1, b1)


def _mm2_kernel(a_ref, b_ref, w_ref, bias_ref, oa, ob):
    h = jnp.concatenate([a_ref[...], b_ref[...]], axis=1)
    h = jax.nn.relu(h)
    o = jnp.dot(h, w_ref[...], preferred_element_type=jnp.float32) + bias_ref[...]
    half = o.shape[1] // 2
    oa[...] = o[:, :half]
    ob[...] = o[:, half:]


def _mm2(h1a, h1b, W2, b2):
    blk = 2048
    m = W2.shape[1]
    half = m // 2
    sds = jax.ShapeDtypeStruct((NP, half), jnp.float32)
    return pl.pallas_call(
        _mm2_kernel,
        grid=(NP // blk,),
        in_specs=[
            pl.BlockSpec((blk, 64), lambda i: (i, 0)),
            pl.BlockSpec((blk, 64), lambda i: (i, 0)),
            pl.BlockSpec((128, m), lambda i: (0, 0)),
            pl.BlockSpec((m,), lambda i: (0,)),
        ],
        out_specs=[pl.BlockSpec((blk, half), lambda i: (i, 0))] * 2,
        out_shape=[sds, sds],
    )(h1a, h1b, W2, b2)


def kernel(x, hyperedge_index, r, W1, b1, W2, b2):
    node_idx = hyperedge_index[0]
    he_idx = hyperedge_index[1]
    s = _matvec(x, r)
    s_pad = jnp.pad(s, (0, NP - N_NODES_C))
    mxP, mnP = _gb_a(s_pad, node_idx, he_idx)
    uAP, uBP = _gb_b(s_pad, node_idx, he_idx, mxP, mnP)
    ro, ri, w = _gb_c(uAP, uBP)
    ro3 = ro.reshape(16, NCH, CH)
    ri3 = ri.reshape(16, NCH, CH)

    xp = jnp.pad(x, ((0, NP - N_NODES_C), (0, 0)))
    ha, hb = _mm1(xp, W1, b1)
    o1a, o1b = _spmm64(ha, hb, ro3, ri3, w)
    oa, ob = _mm2(o1a, o1b, W2, b2)
    qa, qb = _spmm32(oa, ob, ro3, ri3, w)
    return jnp.concatenate([qa[:N_NODES_C], qb[:N_NODES_C]], axis=1)


# speedup vs baseline: 31.5713x; 1.0057x over previous
import functools

import jax
import jax.numpy as jnp
from jax import lax
from jax.experimental import pallas as pl
from jax.experimental.pallas import tpu as pltpu
from jax.experimental.pallas import tpu_sc as plsc

N_NODES_C = 10000
N_HE_C = 10000
NNZ_C = 320000
NP = 10240
ST = 640
EPH2 = NNZ_C // 32
UPT = 3 * ST
NUPD = 16 * UPT
CH = 128
NCH = UPT // CH
NCH_P1 = 10
BIG = N_NODES_C
NEGF = -3.0e38
POSF = 3.0e38

_mesh = plsc.VectorSubcoreMesh(core_axis_name="c", subcore_axis_name="s")


def _mm_kernel(x_ref, w_ref, b_ref, o_ref):
    o_ref[...] = jnp.dot(x_ref[...], w_ref[...],
                         preferred_element_type=jnp.float32) + b_ref[...]


def _matmul_bias(x, w, b):
    n, k = x.shape
    m = w.shape[1]
    blk = 2000
    return pl.pallas_call(
        _mm_kernel,
        grid=(n // blk,),
        in_specs=[
            pl.BlockSpec((blk, k), lambda i: (i, 0)),
            pl.BlockSpec((k, m), lambda i: (0, 0)),
            pl.BlockSpec((m,), lambda i: (0,)),
        ],
        out_specs=pl.BlockSpec((blk, m), lambda i: (i, 0)),
        out_shape=jax.ShapeDtypeStruct((n, m), jnp.float32),
    )(x, w, b)


def _matvec_kernel(x_ref, r_ref, o_ref):
    o_ref[...] = jnp.dot(x_ref[...], r_ref[...],
                         preferred_element_type=jnp.float32)


def _matvec(x, r):
    n, k = x.shape
    blk = 2000
    return pl.pallas_call(
        _matvec_kernel,
        grid=(n // blk,),
        in_specs=[
            pl.BlockSpec((blk, k), lambda i: (i, 0)),
            pl.BlockSpec((k, 128), lambda i: (0, 0)),
        ],
        out_specs=pl.BlockSpec((blk, 128), lambda i: (i, 0)),
        out_shape=jax.ShapeDtypeStruct((n, 128), jnp.float32),
    )(x, jnp.tile(r[:, None], (1, 128)))[:, 0]


def _fill(ref, nwords, val, dtype):
    vec = jnp.full((16,), val, dtype)

    def body(i, _):
        ref[pl.ds(i * 16, 16)] = vec
        return 0

    lax.fori_loop(0, nwords // 16, body, 0)


def _winner_rmw(conflict_ref, idx, mask0, lane, updates):
    def cond(pend):
        return jnp.any(pend)

    def body(pend):
        plsc.store_scatter(conflict_ref, [idx], lane, mask=pend)
        win = plsc.load_gather(conflict_ref, [idx], mask=pend)
        wm = pend & (win == lane)
        for ref, val, comb in updates:
            cur = plsc.load_gather(ref, [idx], mask=wm)
            plsc.store_scatter(ref, [idx], comb(cur, val), mask=wm)
        return pend & jnp.logical_not(wm)

    lax.while_loop(cond, body, mask0)


def _combine_stripe(sh, stf, t, op, init, nrefs=16, sem=None):
    if sem is not None:
        copies = [
            pltpu.async_copy(sh.at[k, pl.ds(t * ST, ST)], stf.at[k], sem)
            for k in range(nrefs)
        ]
        for cp in copies:
            cp.wait()
    else:
        for k in range(nrefs):
            pltpu.sync_copy(sh.at[k, pl.ds(t * ST, ST)], stf.at[k])

    def make_body(out_ref):
        def body(j, _):
            acc = jnp.full((16,), init)
            for k in range(nrefs):
                acc = op(acc, stf[k, pl.ds(j * 16, 16)])
            out_ref[pl.ds(j * 16, 16)] = acc
            return 0
        return body

    return make_body


def _gba_work(s_hbm, nidx_hbm, hidx_hbm, mxP_hbm, mnP_hbm,
              s_tab, idx_n, idx_h, segA, segB, conflict, stf, strb, shF,
              sem, c, t):
    g = c * 16 + t
    lane = lax.iota(jnp.int32, 16)
    full = jnp.full((16,), True)

    pltpu.sync_copy(s_hbm, s_tab)
    pltpu.sync_copy(nidx_hbm.at[pl.ds(g * EPH2, EPH2)], idx_n)
    pltpu.sync_copy(hidx_hbm.at[pl.ds(g * EPH2, EPH2)], idx_h)
    _fill(segA, NP, NEGF, jnp.float32)
    _fill(segB, NP, POSF, jnp.float32)

    def phaseB(i, _):
        hv = idx_h[pl.ds(i * 16, 16)]
        nv = idx_n[pl.ds(i * 16, 16)]
        sv = plsc.load_gather(s_tab, [nv])
        _winner_rmw(conflict, hv, full, lane,
                    [(segA, sv, jnp.maximum), (segB, sv, jnp.minimum)])
        return 0

    lax.fori_loop(0, EPH2 // 16, phaseB, 0, unroll=4)

    pltpu.sync_copy(segA, shF.at[t])
    plsc.subcore_barrier()
    body = _combine_stripe(shF, stf, t, jnp.maximum, NEGF, sem=sem)(strb)
    lax.fori_loop(0, ST // 16, body, 0)
    pltpu.sync_copy(strb, mxP_hbm.at[c, pl.ds(t * ST, ST)])
    plsc.subcore_barrier()
    pltpu.sync_copy(segB, shF.at[t])
    plsc.subcore_barrier()
    body = _combine_stripe(shF, stf, t, jnp.minimum, POSF, sem=sem)(strb)
    lax.fori_loop(0, ST // 16, body, 0)
    pltpu.sync_copy(strb, mnP_hbm.at[c, pl.ds(t * ST, ST)])


def _gba_body(s_hbm, nidx_hbm, hidx_hbm, mxP_hbm, mnP_hbm, *scratch):
    c = lax.axis_index("c")
    t = lax.axis_index("s")
    _gba_work(s_hbm, nidx_hbm, hidx_hbm, mxP_hbm, mnP_hbm, *scratch, c, t)


_gb_a = functools.partial(
    pl.kernel,
    out_type=[
        jax.ShapeDtypeStruct((2, NP), jnp.float32),
        jax.ShapeDtypeStruct((2, NP), jnp.float32),
    ],
    mesh=_mesh,
    compiler_params=pltpu.CompilerParams(needs_layout_passes=False),
    scratch_types=[
        pltpu.VMEM((NP,), jnp.float32),
        pltpu.VMEM((EPH2,), jnp.int32),
        pltpu.VMEM((EPH2,), jnp.int32),
        pltpu.VMEM((NP,), jnp.float32),
        pltpu.VMEM((NP,), jnp.float32),
        pltpu.VMEM((NP,), jnp.int32),
        pltpu.VMEM((16, ST), jnp.float32),
        pltpu.VMEM((ST,), jnp.float32),
        pltpu.VMEM_SHARED((16, NP), jnp.float32),
        pltpu.SemaphoreType.DMA,
    ],
)(_gba_body)


def _elemwise2(dst, other, n, op):
    def body(i, _):
        sl = pl.ds(i * 16, 16)
        dst[sl] = op(dst[sl], other[sl])
        return 0

    lax.fori_loop(0, n // 16, body, 0)


def _gbb_work(s_hbm, nidx_hbm, hidx_hbm, mxP_hbm, mnP_hbm, uAP_hbm, uBP_hbm,
              s_tab, idx_n, idx_h, segA, segB, tmp, uA, uB, conflict,
              sti, strb, shI, sem, c, t):
    g = c * 16 + t
    lane = lax.iota(jnp.int32, 16)
    full = jnp.full((16,), True)

    pltpu.sync_copy(s_hbm, s_tab)
    pltpu.sync_copy(nidx_hbm.at[pl.ds(g * EPH2, EPH2)], idx_n)
    pltpu.sync_copy(hidx_hbm.at[pl.ds(g * EPH2, EPH2)], idx_h)
    pltpu.sync_copy(mxP_hbm.at[0], segA)
    pltpu.sync_copy(mxP_hbm.at[1], tmp)
    _elemwise2(segA, tmp, NP, jnp.maximum)
    pltpu.sync_copy(mnP_hbm.at[0], segB)
    pltpu.sync_copy(mnP_hbm.at[1], tmp)
    _elemwise2(segB, tmp, NP, jnp.minimum)
    _fill(uA, NP, BIG, jnp.int32)
    _fill(uB, NP, BIG, jnp.int32)

    def phaseC(i, _):
        hv = idx_h[pl.ds(i * 16, 16)]
        nv = idx_n[pl.ds(i * 16, 16)]
        sv = plsc.load_gather(s_tab, [nv])
        mx = plsc.load_gather(segA, [hv])
        mn = plsc.load_gather(segB, [hv])
        cand_hi = jnp.where(sv == mx, nv, BIG)
        cand_lo = jnp.where(sv == mn, nv, BIG)
        _winner_rmw(conflict, hv, full, lane,
                    [(uA, cand_hi, jnp.minimum), (uB, cand_lo, jnp.minimum)])
        return 0

    lax.fori_loop(0, EPH2 // 16, phaseC, 0, unroll=4)

    pltpu.sync_copy(uA, shI.at[t])
    plsc.subcore_barrier()
    body = _combine_stripe(shI, sti, t, jnp.minimum, BIG, sem=sem)(strb)
    lax.fori_loop(0, ST // 16, body, 0)
    pltpu.sync_copy(strb, uAP_hbm.at[c, pl.ds(t * ST, ST)])
    plsc.subcore_barrier()
    pltpu.sync_copy(uB, shI.at[t])
    plsc.subcore_barrier()
    body = _combine_stripe(shI, sti, t, jnp.minimum, BIG, sem=sem)(strb)
    lax.fori_loop(0, ST // 16, body, 0)
    pltpu.sync_copy(strb, uBP_hbm.at[c, pl.ds(t * ST, ST)])


def _gbb_body(s_hbm, nidx_hbm, hidx_hbm, mxP_hbm, mnP_hbm,
              uAP_hbm, uBP_hbm, *scratch):
    c = lax.axis_index("c")
    t = lax.axis_index("s")
    _gbb_work(s_hbm, nidx_hbm, hidx_hbm, mxP_hbm, mnP_hbm, uAP_hbm, uBP_hbm,
              *scratch, c, t)


_gb_b = functools.partial(
    pl.kernel,
    out_type=[
        jax.ShapeDtypeStruct((2, NP), jnp.int32),
        jax.ShapeDtypeStruct((2, NP), jnp.int32),
    ],
    mesh=_mesh,
    compiler_params=pltpu.CompilerParams(needs_layout_passes=False),
    scratch_types=[
        pltpu.VMEM((NP,), jnp.float32),
        pltpu.VMEM((EPH2,), jnp.int32),
        pltpu.VMEM((EPH2,), jnp.int32),
        pltpu.VMEM((NP,), jnp.float32),
        pltpu.VMEM((NP,), jnp.float32),
        pltpu.VMEM((NP,), jnp.float32),
        pltpu.VMEM((NP,), jnp.int32),
        pltpu.VMEM((NP,), jnp.int32),
        pltpu.VMEM((NP,), jnp.int32),
        pltpu.VMEM((16, ST), jnp.int32),
        pltpu.VMEM((ST,), jnp.int32),
        pltpu.VMEM_SHARED((16, NP), jnp.int32),
        pltpu.SemaphoreType.DMA,
    ],
)(_gbb_body)


def _gbc_work(uAP_hbm, uBP_hbm, ro_hbm, ri_hbm, w_hbm,
              s_tab, segA, conflict, stf, ustrA, ustrB, u2,
              srcb, dstb, diagb, dinvstr, wbuf, shF):
    t = lax.axis_index("s")
    lane = lax.iota(jnp.int32, 16)

    stripe = pl.ds(t * ST, ST)
    pltpu.sync_copy(uAP_hbm.at[0, stripe], ustrA)
    pltpu.sync_copy(uAP_hbm.at[1, stripe], u2)
    _elemwise2(ustrA, u2, ST, jnp.minimum)
    pltpu.sync_copy(uBP_hbm.at[0, stripe], ustrB)
    pltpu.sync_copy(uBP_hbm.at[1, stripe], u2)
    _elemwise2(ustrB, u2, ST, jnp.minimum)

    def phaseD1(j, _):
        ua = ustrA[pl.ds(j * 16, 16)]
        ub = ustrB[pl.ds(j * 16, 16)]
        valid = (ua < BIG) & (ub < BIG) & (ua != ub)
        srcb[pl.ds(j * 16, 16)] = jnp.where(valid, ua, 0)
        dstb[pl.ds(j * 16, 16)] = jnp.where(valid, ub, 0)
        return 0

    lax.fori_loop(0, ST // 16, phaseD1, 0)

    _fill(segA, NP, jnp.float32(0.0), jnp.float32)
    onef = jnp.full((16,), 1.0, jnp.float32)

    def phaseD2(j, _):
        ua = ustrA[pl.ds(j * 16, 16)]
        ub = ustrB[pl.ds(j * 16, 16)]
        sv16 = srcb[pl.ds(j * 16, 16)]
        dv16 = dstb[pl.ds(j * 16, 16)]
        valid = (ua < BIG) & (ub < BIG) & (ua != ub)
        _winner_rmw(conflict, sv16, valid, lane,
                    [(segA, onef, lambda c, v: c + v)])
        _winner_rmw(conflict, dv16, valid, lane,
                    [(segA, onef, lambda c, v: c + v)])
        return 0

    lax.fori_loop(0, ST // 16, phaseD2, 0)

    pltpu.sync_copy(segA, shF.at[t])
    plsc.subcore_barrier()

    def degbody(j, _):
        acc = jnp.full((16,), 0.0, jnp.float32)
        for k in range(16):
            acc = acc + stf[k, pl.ds(j * 16, 16)]
        deg = acc + 1.0
        bits = plsc.bitcast(deg, jnp.int32)
        y = plsc.bitcast(jnp.int32(0x5F3759DF) - (bits >> 1), jnp.float32)
        for _i in range(3):
            y = y * (1.5 - 0.5 * deg * y * y)
        dinvstr[pl.ds(j * 16, 16)] = y
        return 0

    lax.fori_loop(0, ST // 16, degbody, 0)
    pltpu.sync_copy(dinvstr, shF.at[0, pl.ds(t * ST, ST)])
    plsc.subcore_barrier()
    pltpu.sync_copy(shF.at[0], s_tab)
    plsc.subcore_barrier()

    def dsbody(j, _):
        y = dinvstr[pl.ds(j * 16, 16)]
        wbuf[pl.ds(2 * ST + j * 16, 16)] = y * y
        diagb[pl.ds(j * 16, 16)] = t * ST + j * 16 + lane
        return 0

    lax.fori_loop(0, ST // 16, dsbody, 0)

    def wbody(j, _):
        ua = ustrA[pl.ds(j * 16, 16)]
        ub = ustrB[pl.ds(j * 16, 16)]
        sv16 = srcb[pl.ds(j * 16, 16)]
        dv16 = dstb[pl.ds(j * 16, 16)]
        valid = (ua < BIG) & (ub < BIG) & (ua != ub)
        ds_ = plsc.load_gather(s_tab, [sv16])
        dd_ = plsc.load_gather(s_tab, [dv16])
        wv = jnp.where(valid, ds_ * dd_, 0.0)
        wbuf[pl.ds(j * 16, 16)] = wv
        wbuf[pl.ds(ST + j * 16, 16)] = wv
        return 0

    lax.fori_loop(0, ST // 16, wbody, 0)

    pltpu.sync_copy(srcb, ro_hbm.at[pl.ds(t * UPT, ST)])
    pltpu.sync_copy(dstb, ro_hbm.at[pl.ds(t * UPT + ST, ST)])
    pltpu.sync_copy(diagb, ro_hbm.at[pl.ds(t * UPT + 2 * ST, ST)])
    pltpu.sync_copy(dstb, ri_hbm.at[pl.ds(t * UPT, ST)])
    pltpu.sync_copy(srcb, ri_hbm.at[pl.ds(t * UPT + ST, ST)])
    pltpu.sync_copy(diagb, ri_hbm.at[pl.ds(t * UPT + 2 * ST, ST)])
    pltpu.sync_copy(wbuf, w_hbm.at[pl.ds(t * UPT, UPT)])


def _gbc_body(uAP_hbm, uBP_hbm, ro_hbm, ri_hbm, w_hbm, *scratch):
    c = lax.axis_index("c")
    pl.when(c == 0)(lambda: _gbc_work(
        uAP_hbm, uBP_hbm, ro_hbm, ri_hbm, w_hbm, *scratch))


_gb_c = functools.partial(
    pl.kernel,
    out_type=[
        jax.ShapeDtypeStruct((NUPD,), jnp.int32),
        jax.ShapeDtypeStruct((NUPD,), jnp.int32),
        jax.ShapeDtypeStruct((NUPD,), jnp.float32),
    ],
    mesh=_mesh,
    compiler_params=pltpu.CompilerParams(needs_layout_passes=False),
    scratch_types=[
        pltpu.VMEM((NP,), jnp.float32),
        pltpu.VMEM((NP,), jnp.float32),
        pltpu.VMEM((NP,), jnp.int32),
        pltpu.VMEM((16, ST), jnp.float32),
        pltpu.VMEM((ST,), jnp.int32),
        pltpu.VMEM((ST,), jnp.int32),
        pltpu.VMEM((ST,), jnp.int32),
        pltpu.VMEM((ST,), jnp.int32),
        pltpu.VMEM((ST,), jnp.int32),
        pltpu.VMEM((ST,), jnp.int32),
        pltpu.VMEM((ST,), jnp.float32),
        pltpu.VMEM((UPT,), jnp.float32),
        pltpu.VMEM_SHARED((16, NP), jnp.float32),
    ],
)(_gbc_body)


def _spmm_work(h_hbm, o_hbm, ro_hbm, ri_hbm, w_hbm,
               gbuf, rov, riv, wv, acc, semg, sems, t, f2):
    stripe = pl.ds(t * ST, ST)
    pltpu.sync_copy(ro_hbm.at[t], rov)
    pltpu.sync_copy(ri_hbm.at[t], riv)
    pltpu.sync_copy(w_hbm.at[pl.ds(t * UPT, UPT)], wv)

    def make_scale(base):
        def scale16(jj, _):
            w16 = wv[pl.ds(jj * 16, 16)]
            for k16 in range(16):
                sc = w16[k16]
                row = jj * 16 + k16 - base * CH
                for cc in range(f2 // 16):
                    col = pl.ds(cc * 16, 16)
                    gbuf[row, col] = gbuf[row, col] * sc
            return 0
        return scale16

    def run_pass(chunks, add):
        base = chunks[0]
        gathers = [
            pltpu.async_copy(h_hbm.at[riv.at[chunk]],
                             gbuf.at[pl.ds((chunk - base) * CH, CH)], semg)
            for chunk in chunks
        ]
        for g in gathers:
            g.wait()
        lax.fori_loop(base * CH // 16, (chunks[-1] + 1) * CH // 16,
                      make_scale(base), 0)
        scatters = [
            pltpu.async_copy(gbuf.at[pl.ds((chunk - base) * CH, CH)],
                             acc.at[rov.at[chunk]], sems, add=add)
            for chunk in chunks
        ]
        for sctr in scatters:
            sctr.wait()

    run_pass(list(range(2 * ST // CH, NCH)), add=False)
    plsc.subcore_barrier()
    run_pass(list(range(0, 2 * ST // CH)), add=True)

    plsc.subcore_barrier()
    pltpu.sync_copy(acc.at[stripe], o_hbm.at[stripe])


def _make_spmm(f2):
    def body(ha, hb, ro3, ri3, w, oa, ob,
             gbuf, rov, riv, wv, acc, semg, sems):
        c = lax.axis_index("c")
        t = lax.axis_index("s")
        pl.when(c == 0)(lambda: _spmm_work(
            ha, oa, ro3, ri3, w, gbuf, rov, riv, wv, acc, semg, sems, t, f2))
        pl.when(c == 1)(lambda: _spmm_work(
            hb, ob, ro3, ri3, w, gbuf, rov, riv, wv, acc, semg, sems, t, f2))

    return functools.partial(
        pl.kernel,
        out_type=[
            jax.ShapeDtypeStruct((NP, f2), jnp.float32),
            jax.ShapeDtypeStruct((NP, f2), jnp.float32),
        ],
        mesh=_mesh,
        compiler_params=pltpu.CompilerParams(needs_layout_passes=False,
                                             use_tc_tiling_on_sc=False),
        scratch_types=[
            pltpu.VMEM((NCH_P1 * CH, f2), jnp.float32),
            pltpu.VMEM((NCH, CH), jnp.int32),
            pltpu.VMEM((NCH, CH), jnp.int32),
            pltpu.VMEM((UPT,), jnp.float32),
            pltpu.VMEM_SHARED((NP, f2), jnp.float32),
            pltpu.SemaphoreType.DMA,
            pltpu.SemaphoreType.DMA,
        ],
    )(body)


_spmm64 = _make_spmm(64)
_spmm32 = _make_spmm(32)


def _mm_split_kernel(x_ref, w_ref, b_ref, ha, hb):
    h = jnp.dot(x_ref[...], w_ref[...],
                preferred_element_type=jnp.float32) + b_ref[...]
    half = h.shape[1] // 2
    ha[...] = h[:, :half]
    hb[...] = h[:, half:]


def _mm1(xp, W1, b1):
    blk = 2048
    m = W1.shape[1]
    half = m // 2
    sds = jax.ShapeDtypeStruct((NP, half), jnp.float32)
    return pl.pallas_call(
        _mm_split_kernel,
        grid=(NP // blk,),
        in_specs=[
            pl.BlockSpec((blk, 128), lambda i: (i, 0)),
            pl.BlockSpec((128, m), lambda i: (0, 0)),
            pl.BlockSpec((m,), lambda i: (0,)),
        ],
        out_specs=[pl.BlockSpec((blk, half), lambda i: (i, 0))] * 2,
        out_shape=[sds, sds],
    )(xp, W1, b1)


def _mm2_kernel(a_ref, b_ref, w_ref, bias_ref, oa, ob):
    h = jnp.concatenate([a_ref[...], b_ref[...]], axis=1)
    h = jax.nn.relu(h)
    o = jnp.dot(h, w_ref[...], preferred_element_type=jnp.float32) + bias_ref[...]
    half = o.shape[1] // 2
    oa[...] = o[:, :half]
    ob[...] = o[:, half:]


def _mm2(h1a, h1b, W2, b2):
    blk = 2048
    m = W2.shape[1]
    half = m // 2
    sds = jax.ShapeDtypeStruct((NP, half), jnp.float32)
    return pl.pallas_call(
        _mm2_kernel,
        grid=(NP // blk,),
        in_specs=[
            pl.BlockSpec((blk, 64), lambda i: (i, 0)),
            pl.BlockSpec((blk, 64), lambda i: (i, 0)),
            pl.BlockSpec((128, m), lambda i: (0, 0)),
            pl.BlockSpec((m,), lambda i: (0,)),
        ],
        out_specs=[pl.BlockSpec((blk, half), lambda i: (i, 0))] * 2,
        out_shape=[sds, sds],
    )(h1a, h1b, W2, b2)


def kernel(x, hyperedge_index, r, W1, b1, W2, b2):
    node_idx = hyperedge_index[0]
    he_idx = hyperedge_index[1]
    s = _matvec(x, r)
    s_pad = jnp.pad(s, (0, NP - N_NODES_C))
    mxP, mnP = _gb_a(s_pad, node_idx, he_idx)
    uAP, uBP = _gb_b(s_pad, node_idx, he_idx, mxP, mnP)
    ro, ri, w = _gb_c(uAP, uBP)
    ro3 = ro.reshape(16, NCH, CH)
    ri3 = ri.reshape(16, NCH, CH)

    xp = jnp.pad(x, ((0, NP - N_NODES_C), (0, 0)))
    ha, hb = _mm1(xp, W1, b1)
    o1a, o1b = _spmm64(ha, hb, ro3, ri3, w)
    oa, ob = _mm2(o1a, o1b, W2, b2)
    qa, qb = _spmm32(oa, ob, ro3, ri3, w)
    return jnp.concatenate([qa[:N_NODES_C], qb[:N_NODES_C]], axis=1)
